# Initial kernel scaffold; baseline (speedup 1.0000x reference)
import jax
import jax.numpy as jnp
from jax.experimental import pallas as pl


def kernel(x, edge_index, edge_attr, params):
    raise NotImplementedError("write your pallas kernel here")


# speedup vs baseline: 2.4637x; 2.4637x over previous
import functools

import jax
import jax.numpy as jnp
from jax import lax
from jax.experimental import pallas as pl
from jax.experimental.pallas import tpu as pltpu
from jax.experimental.pallas import tpu_sc as plsc

_N = 10000
_E = 160000
_D = 128
_NC = 2
_NS = 16
_NW = _NC * _NS
_RPW = _E // _NW
_CH = 128
_NFULL = _RPW // _CH
_TAIL = _RPW - _NFULL * _CH
_NPAD = 10240
_RPT = _NPAD // _NS


def _sc_mesh():
    return plsc.VectorSubcoreMesh(core_axis_name="c", subcore_axis_name="s",
                                  num_cores=_NC, num_subcores=_NS)


def _sc_gather(tables, idxs, idx_of):
    n_t = len(tables)
    n_i = len(idxs)
    scratch = (
        [pltpu.VMEM((_CH,), jnp.int32) for _ in range(n_i)]
        + [pltpu.VMEM((_CH, _D), jnp.float32) for _ in range(n_t)]
        + [pltpu.VMEM((_TAIL,), jnp.int32) for _ in range(n_i)]
        + [pltpu.VMEM((_TAIL, _D), jnp.float32) for _ in range(n_t)]
        + [pltpu.SemaphoreType.DMA]
    )
    out_type = tuple(
        jax.ShapeDtypeStruct((_E, _D), jnp.float32) for _ in range(n_t))

    def body(*refs):
        tab = refs[:n_t]
        idx = refs[n_t:n_t + n_i]
        outs = refs[n_t + n_i:n_t + n_i + n_t]
        s = n_t + n_i + n_t
        idx_b = refs[s:s + n_i]; s += n_i
        row_b = refs[s:s + n_t]; s += n_t
        idx_t = refs[s:s + n_i]; s += n_i
        row_t = refs[s:s + n_t]; s += n_t
        sem = refs[s]
        wid = lax.axis_index("s") * _NC + lax.axis_index("c")
        base = wid * _RPW

        def do(off, ib, rb, nrows):
            for j in range(n_i):
                pltpu.sync_copy(idx[j].at[pl.ds(off, nrows)], ib[j])
            cps = [pltpu.async_copy(tab[t].at[ib[idx_of[t]]], rb[t], sem)
                   for t in range(n_t)]
            for c in cps:
                c.wait()
            for t in range(n_t):
                pltpu.sync_copy(rb[t], outs[t].at[pl.ds(off, nrows)])

        def loop_body(g, carry):
            do(base + g * _CH, idx_b, row_b, _CH)
            return carry

        lax.fori_loop(0, _NFULL, loop_body, 0)
        do(base + _NFULL * _CH, idx_t, row_t, _TAIL)

    f = pl.kernel(body, out_type=out_type, mesh=_sc_mesh(),
                  scratch_types=scratch)
    return f(*tables, *idxs)


def _sc_scatter_add(msg, dst, zrows):
    scratch = [
        pltpu.VMEM((_CH,), jnp.int32),
        pltpu.VMEM((_CH, _D), jnp.float32),
        pltpu.VMEM((_TAIL,), jnp.int32),
        pltpu.VMEM((_TAIL, _D), jnp.float32),
        pltpu.VMEM_SHARED((_NPAD, _D), jnp.float32),
    ]

    def body(msg_hbm, dst_hbm, z_hbm, out_hbm, idx_b, row_b, idx_t, row_t, acc):
        cid = lax.axis_index("c")
        sid = lax.axis_index("s")
        wid = sid * _NC + cid
        base = wid * _RPW
        pltpu.sync_copy(z_hbm, acc.at[pl.ds(sid * _RPT, _RPT)])
        plsc.subcore_barrier()

        def do(off, ib, rb):
            pltpu.sync_copy(dst_hbm.at[pl.ds(off, ib.shape[0])], ib)
            pltpu.sync_copy(msg_hbm.at[pl.ds(off, rb.shape[0])], rb)
            pltpu.sync_copy(rb, acc.at[ib], add=True)

        def loop_body(g, carry):
            do(base + g * _CH, idx_b, row_b)
            return carry

        lax.fori_loop(0, _NFULL, loop_body, 0)
        do(base + _NFULL * _CH, idx_t, row_t)
        plsc.subcore_barrier()
        pltpu.sync_copy(acc.at[pl.ds(sid * _RPT, _RPT)],
                        out_hbm.at[cid, pl.ds(sid * _RPT, _RPT)])

    f = pl.kernel(body,
                  out_type=jax.ShapeDtypeStruct((_NC, _NPAD, _D), jnp.float32),
                  mesh=_sc_mesh(), scratch_types=scratch)
    return f(msg, dst, zrows)


def _linear(x, wt, b=None, act=None, br=2000):
    r, d = x.shape
    f_out = wt.shape[1]
    assert r % br == 0
    grid = (r // br,)
    in_specs = [pl.BlockSpec((br, d), lambda i: (i, 0)),
                pl.BlockSpec((d, f_out), lambda i: (0, 0))]
    args = [x, wt]
    if b is not None:
        in_specs.append(pl.BlockSpec((1, f_out), lambda i: (0, 0)))
        args.append(b.reshape(1, f_out))

    def body(x_ref, w_ref, *rest):
        if b is not None:
            b_ref, o_ref = rest
        else:
            (o_ref,) = rest
        y = jnp.dot(x_ref[...], w_ref[...], preferred_element_type=jnp.float32)
        if b is not None:
            y = y + b_ref[...]
        if act == "relu":
            y = jnp.maximum(y, 0.0)
        o_ref[...] = y

    return pl.pallas_call(
        body, grid=grid, in_specs=in_specs,
        out_specs=pl.BlockSpec((br, f_out), lambda i: (i, 0)),
        out_shape=jax.ShapeDtypeStruct((r, f_out), jnp.float32))(*args)


def _edge_msg(kd, qs, e, vs, br=2000):
    grid = (_E // br,)
    spec = pl.BlockSpec((br, _D), lambda i: (i, 0))

    def body(kd_r, qs_r, e_r, vs_r, o_r):
        o_r[...] = jax.nn.sigmoid(kd_r[...] + qs_r[...] + e_r[...]) * vs_r[...]

    return pl.pallas_call(
        body, grid=grid, in_specs=[spec] * 4, out_specs=spec,
        out_shape=jax.ShapeDtypeStruct((_E, _D), jnp.float32))(kd, qs, e, vs)


def _atom_post(x, skip_o, agg, gamma, beta):
    def body(x_r, s_r, a_r, g_r, b_r, o_r):
        h = s_r[...] + a_r[0, :_N] + a_r[1, :_N]
        mu = jnp.mean(h, axis=0, keepdims=True)
        var = jnp.mean((h - mu) ** 2, axis=0, keepdims=True)
        hn = g_r[...] * (h - mu) * lax.rsqrt(var + 1e-5) + b_r[...]
        o_r[...] = x_r[...] + jnp.maximum(hn, 0.0)

    return pl.pallas_call(
        body, out_shape=jax.ShapeDtypeStruct((_N, _D), jnp.float32))(
            x, skip_o, agg, gamma.reshape(1, _D), beta.reshape(1, _D))


def _bond_sum_stats(e0, a1g, a2g, br=2000):
    grid = (_E // br,)
    spec = pl.BlockSpec((br, _D), lambda i: (i, 0))

    def body(e0_r, a1_r, a2_r, s_r, st_r, sacc, qacc):
        i = pl.program_id(0)
        v = e0_r[...] + a1_r[...] + a2_r[...]
        s_r[...] = v

        @pl.when(i == 0)
        def _():
            sacc[...] = jnp.zeros_like(sacc)
            qacc[...] = jnp.zeros_like(qacc)

        sacc[...] += jnp.sum(v, axis=0).reshape(1, _D)
        qacc[...] += jnp.sum(v * v, axis=0).reshape(1, _D)

        @pl.when(i == grid[0] - 1)
        def _():
            st_r[0:1, :] = sacc[...]
            st_r[1:2, :] = qacc[...]

    return pl.pallas_call(
        body, grid=grid, in_specs=[spec] * 3,
        out_specs=(spec, pl.BlockSpec((2, _D), lambda i: (0, 0))),
        out_shape=(jax.ShapeDtypeStruct((_E, _D), jnp.float32),
                   jax.ShapeDtypeStruct((2, _D), jnp.float32)),
        scratch_shapes=[pltpu.VMEM((1, _D), jnp.float32),
                        pltpu.VMEM((1, _D), jnp.float32)])(e0, a1g, a2g)


def _bond_post(ea, s, stats, gamma, beta, br=2000):
    grid = (_E // br,)
    spec = pl.BlockSpec((br, _D), lambda i: (i, 0))
    one = pl.BlockSpec((1, _D), lambda i: (0, 0))

    def body(ea_r, s_r, st_r, g_r, b_r, o_r):
        mu = st_r[0:1, :] * (1.0 / _E)
        var = st_r[1:2, :] * (1.0 / _E) - mu * mu
        hn = g_r[...] * (s_r[...] - mu) * lax.rsqrt(var + 1e-5) + b_r[...]
        o_r[...] = ea_r[...] + jnp.maximum(hn, 0.0)

    return pl.pallas_call(
        body, grid=grid,
        in_specs=[spec, spec, pl.BlockSpec((2, _D), lambda i: (0, 0)), one, one],
        out_specs=spec,
        out_shape=jax.ShapeDtypeStruct((_E, _D), jnp.float32))(
            ea, s, stats, gamma.reshape(1, _D), beta.reshape(1, _D))


def _atom_head(x, w1t, b1, w2t, b2):
    f_out = w2t.shape[1]

    def body(x_r, w1_r, b1_r, w2_r, b2_r, o_r):
        z = jnp.mean(x_r[...], axis=0, keepdims=True)
        h = jnp.maximum(
            jnp.dot(z, w1_r[...], preferred_element_type=jnp.float32)
            + b1_r[...], 0.0)
        o_r[...] = jnp.dot(h, w2_r[...],
                           preferred_element_type=jnp.float32) + b2_r[...]

    return pl.pallas_call(
        body, out_shape=jax.ShapeDtypeStruct((1, f_out), jnp.float32))(
            x, w1t, b1.reshape(1, -1), w2t, b2.reshape(1, -1))


def kernel(x, edge_index, edge_attr, params):
    src = edge_index[0]
    dst = edge_index[1]
    zrows = jnp.zeros((_RPT, _D), jnp.float32)

    for pa, pb in zip(params["atom_layers"], params["bond_layers"]):
        wkqvs = jnp.concatenate(
            [pa["key"]["W"], pa["query"]["W"], pa["value"]["W"],
             pa["skip"]["W"]], axis=0).T
        bkqvs = jnp.concatenate(
            [jnp.zeros((_D,), jnp.float32), jnp.zeros((_D,), jnp.float32),
             pa["value"]["b"], pa["skip"]["b"]])
        kqvs = _linear(x, wkqvs, bkqvs, br=2000)
        kk, qq, vv = kqvs[:, :_D], kqvs[:, _D:2 * _D], kqvs[:, 2 * _D:3 * _D]
        skip_o = kqvs[:, 3 * _D:]
        e_bias = pa["edge"]["b"] + pa["key"]["b"] + pa["query"]["b"]
        e = _linear(edge_attr, pa["edge"]["W"].T, e_bias, br=2000)
        kd, qs, vs = _sc_gather((kk, qq, vv), (dst, src), (0, 1, 1))
        msg = _edge_msg(kd, qs, e, vs)
        agg = _sc_scatter_add(msg, dst, zrows)
        x = _atom_post(x, skip_o, agg, pa["bn_gamma"], pa["bn_beta"])

        w12 = jnp.concatenate([pb["v1"]["W"], pb["v2"]["W"]], axis=0).T
        a12 = _linear(x, w12, None, br=2000)
        a1, a2 = a12[:, :_D], a12[:, _D:]
        b0 = pb["v0"]["b"] + pb["v1"]["b"] + pb["v2"]["b"]
        e0 = _linear(edge_attr, pb["v0"]["W"].T, b0, br=2000)
        a1g, a2g = _sc_gather((a1, a2), (src, dst), (0, 1))
        s, stats = _bond_sum_stats(e0, a1g, a2g)
        edge_attr = _bond_post(edge_attr, s, stats, pb["bn_gamma"],
                               pb["bn_beta"])

    boa = _atom_head(x, params["atom_mlp"]["l1"]["W"].T,
                     params["atom_mlp"]["l1"]["b"],
                     params["atom_mlp"]["l2"]["W"].T,
                     params["atom_mlp"]["l2"]["b"])
    hb = _linear(edge_attr, params["bond_mlp"]["l1"]["W"].T,
                 params["bond_mlp"]["l1"]["b"], act="relu", br=2000)
    bonds = _linear(hb, params["bond_mlp"]["l2"]["W"].T,
                    params["bond_mlp"]["l2"]["b"], br=2000)
    return boa.reshape(-1, 8, 100), bonds


# speedup vs baseline: 2.6723x; 1.0847x over previous
import functools

import jax
import jax.numpy as jnp
from jax import lax
from jax.experimental import pallas as pl
from jax.experimental.pallas import tpu as pltpu
from jax.experimental.pallas import tpu_sc as plsc

_N = 10000
_E = 160000
_D = 128
_NC = 2
_NS = 16
_NW = _NC * _NS
_RPW = _E // _NW
_CH = 128
_NFULL = _RPW // _CH
_TAIL = _RPW - _NFULL * _CH
_NPAD = 10240
_RPT = _NPAD // _NS


def _sc_mesh():
    return plsc.VectorSubcoreMesh(core_axis_name="c", subcore_axis_name="s",
                                  num_cores=_NC, num_subcores=_NS)


_NBUF = 3
_SNBUF = 2
_CHUNKS = [(c * _CH, _CH) for c in range(_NFULL)] + [(_NFULL * _CH, _TAIL)]


def _sc_gather(tables, idxs, idx_of):
    n_t = len(tables)
    n_i = len(idxs)
    scratch = (
        [pltpu.VMEM((_RPW,), jnp.int32) for _ in range(n_i)]
        + [pltpu.VMEM((_CH, _D), jnp.float32) for _ in range(_NBUF)]
        + [pltpu.SemaphoreType.DMA, pltpu.SemaphoreType.DMA]
    )
    out_type = tuple(
        jax.ShapeDtypeStruct((_E, _D), jnp.float32) for _ in range(n_t))

    def body(*refs):
        tab = refs[:n_t]
        idx = refs[n_t:n_t + n_i]
        outs = refs[n_t + n_i:n_t + n_i + n_t]
        s = n_t + n_i + n_t
        ib = refs[s:s + n_i]; s += n_i
        rb = refs[s:s + _NBUF]; s += _NBUF
        sem_g, sem_w = refs[s], refs[s + 1]
        wid = lax.axis_index("s") * _NC + lax.axis_index("c")
        base = wid * _RPW
        for j in range(n_i):
            pltpu.sync_copy(idx[j].at[pl.ds(base, _RPW)], ib[j])

        jobs = [(t, off, sz) for t in range(n_t) for (off, sz) in _CHUNKS]
        pend_w = [None] * _NBUF
        prev = None

        def buf(b, sz):
            return rb[b] if sz == _CH else rb[b].at[pl.ds(0, sz)]

        def flush_prev():
            pb, pt, poff, psz, pg = prev
            pg.wait()
            pend_w[pb] = pltpu.async_copy(
                buf(pb, psz), outs[pt].at[pl.ds(base + poff, psz)], sem_w)

        for c, (t, off, sz) in enumerate(jobs):
            b = c % _NBUF
            if pend_w[b] is not None:
                pend_w[b].wait()
                pend_w[b] = None
            g = pltpu.async_copy(
                tab[t].at[ib[idx_of[t]].at[pl.ds(off, sz)]], buf(b, sz), sem_g)
            if prev is not None:
                flush_prev()
            prev = (b, t, off, sz, g)
        flush_prev()
        for d in pend_w:
            if d is not None:
                d.wait()

    f = pl.kernel(body, out_type=out_type, mesh=_sc_mesh(),
                  scratch_types=scratch)
    return f(*tables, *idxs)


def _sc_scatter_add(msg, dst, zrows):
    scratch = (
        [pltpu.VMEM((_CH,), jnp.int32) for _ in range(_SNBUF)]
        + [pltpu.VMEM((_CH, _D), jnp.float32) for _ in range(_SNBUF)]
        + [pltpu.VMEM((_TAIL,), jnp.int32),
           pltpu.VMEM((_TAIL, _D), jnp.float32),
           pltpu.VMEM_SHARED((_NPAD, _D), jnp.float32),
           pltpu.SemaphoreType.DMA, pltpu.SemaphoreType.DMA]
    )

    def body(msg_hbm, dst_hbm, z_hbm, out_hbm, *rest):
        ib = rest[:_SNBUF]
        rb = rest[_SNBUF:2 * _SNBUF]
        ib_t, rb_t, acc, sem_l, sem_s = rest[2 * _SNBUF:]
        cid = lax.axis_index("c")
        sid = lax.axis_index("s")
        wid = sid * _NC + cid
        base = wid * _RPW
        pltpu.sync_copy(z_hbm, acc.at[pl.ds(sid * _RPT, _RPT)])
        plsc.subcore_barrier()

        pend_s = [None] * (_SNBUF + 1)
        prev = None

        def flush_prev():
            slot, pib, prb, li, lm = prev
            li.wait()
            lm.wait()
            pend_s[slot] = pltpu.async_copy(prb, acc.at[pib], sem_s, add=True)

        for c, (off, sz) in enumerate(_CHUNKS):
            if sz == _CH:
                slot = c % _SNBUF
                cib, crb = ib[slot], rb[slot]
            else:
                slot = _SNBUF
                cib, crb = ib_t, rb_t
            if pend_s[slot] is not None:
                pend_s[slot].wait()
                pend_s[slot] = None
            li = pltpu.async_copy(dst_hbm.at[pl.ds(base + off, sz)], cib, sem_l)
            lm = pltpu.async_copy(msg_hbm.at[pl.ds(base + off, sz)], crb, sem_l)
            if prev is not None:
                flush_prev()
            prev = (slot, cib, crb, li, lm)
        flush_prev()
        for d in pend_s:
            if d is not None:
                d.wait()
        plsc.subcore_barrier()
        pltpu.sync_copy(acc.at[pl.ds(sid * _RPT, _RPT)],
                        out_hbm.at[cid, pl.ds(sid * _RPT, _RPT)])

    f = pl.kernel(body,
                  out_type=jax.ShapeDtypeStruct((_NC, _NPAD, _D), jnp.float32),
                  mesh=_sc_mesh(), scratch_types=scratch)
    return f(msg, dst, zrows)


def _linear(x, wt, b=None, act=None, br=2000):
    r, d = x.shape
    f_out = wt.shape[1]
    assert r % br == 0
    grid = (r // br,)
    in_specs = [pl.BlockSpec((br, d), lambda i: (i, 0)),
                pl.BlockSpec((d, f_out), lambda i: (0, 0))]
    args = [x, wt]
    if b is not None:
        in_specs.append(pl.BlockSpec((1, f_out), lambda i: (0, 0)))
        args.append(b.reshape(1, f_out))

    def body(x_ref, w_ref, *rest):
        if b is not None:
            b_ref, o_ref = rest
        else:
            (o_ref,) = rest
        y = jnp.dot(x_ref[...], w_ref[...], preferred_element_type=jnp.float32)
        if b is not None:
            y = y + b_ref[...]
        if act == "relu":
            y = jnp.maximum(y, 0.0)
        o_ref[...] = y

    return pl.pallas_call(
        body, grid=grid, in_specs=in_specs,
        out_specs=pl.BlockSpec((br, f_out), lambda i: (i, 0)),
        out_shape=jax.ShapeDtypeStruct((r, f_out), jnp.float32))(*args)


def _edge_msg(kd, qs, e, vs, br=2000):
    grid = (_E // br,)
    spec = pl.BlockSpec((br, _D), lambda i: (i, 0))

    def body(kd_r, qs_r, e_r, vs_r, o_r):
        o_r[...] = jax.nn.sigmoid(kd_r[...] + qs_r[...] + e_r[...]) * vs_r[...]

    return pl.pallas_call(
        body, grid=grid, in_specs=[spec] * 4, out_specs=spec,
        out_shape=jax.ShapeDtypeStruct((_E, _D), jnp.float32))(kd, qs, e, vs)


def _atom_post(x, skip_o, agg, gamma, beta):
    def body(x_r, s_r, a_r, g_r, b_r, o_r):
        h = s_r[...] + a_r[0, :_N] + a_r[1, :_N]
        mu = jnp.mean(h, axis=0, keepdims=True)
        var = jnp.mean((h - mu) ** 2, axis=0, keepdims=True)
        hn = g_r[...] * (h - mu) * lax.rsqrt(var + 1e-5) + b_r[...]
        o_r[...] = x_r[...] + jnp.maximum(hn, 0.0)

    return pl.pallas_call(
        body, out_shape=jax.ShapeDtypeStruct((_N, _D), jnp.float32))(
            x, skip_o, agg, gamma.reshape(1, _D), beta.reshape(1, _D))


def _bond_sum_stats(e0, a1g, a2g, br=2000):
    grid = (_E // br,)
    spec = pl.BlockSpec((br, _D), lambda i: (i, 0))

    def body(e0_r, a1_r, a2_r, s_r, st_r, sacc, qacc):
        i = pl.program_id(0)
        v = e0_r[...] + a1_r[...] + a2_r[...]
        s_r[...] = v

        @pl.when(i == 0)
        def _():
            sacc[...] = jnp.zeros_like(sacc)
            qacc[...] = jnp.zeros_like(qacc)

        sacc[...] += jnp.sum(v, axis=0).reshape(1, _D)
        qacc[...] += jnp.sum(v * v, axis=0).reshape(1, _D)

        @pl.when(i == grid[0] - 1)
        def _():
            st_r[0:1, :] = sacc[...]
            st_r[1:2, :] = qacc[...]

    return pl.pallas_call(
        body, grid=grid, in_specs=[spec] * 3,
        out_specs=(spec, pl.BlockSpec((2, _D), lambda i: (0, 0))),
        out_shape=(jax.ShapeDtypeStruct((_E, _D), jnp.float32),
                   jax.ShapeDtypeStruct((2, _D), jnp.float32)),
        scratch_shapes=[pltpu.VMEM((1, _D), jnp.float32),
                        pltpu.VMEM((1, _D), jnp.float32)])(e0, a1g, a2g)


def _bond_post(ea, s, stats, gamma, beta, br=2000):
    grid = (_E // br,)
    spec = pl.BlockSpec((br, _D), lambda i: (i, 0))
    one = pl.BlockSpec((1, _D), lambda i: (0, 0))

    def body(ea_r, s_r, st_r, g_r, b_r, o_r):
        mu = st_r[0:1, :] * (1.0 / _E)
        var = st_r[1:2, :] * (1.0 / _E) - mu * mu
        hn = g_r[...] * (s_r[...] - mu) * lax.rsqrt(var + 1e-5) + b_r[...]
        o_r[...] = ea_r[...] + jnp.maximum(hn, 0.0)

    return pl.pallas_call(
        body, grid=grid,
        in_specs=[spec, spec, pl.BlockSpec((2, _D), lambda i: (0, 0)), one, one],
        out_specs=spec,
        out_shape=jax.ShapeDtypeStruct((_E, _D), jnp.float32))(
            ea, s, stats, gamma.reshape(1, _D), beta.reshape(1, _D))


def _atom_head(x, w1t, b1, w2t, b2):
    f_out = w2t.shape[1]

    def body(x_r, w1_r, b1_r, w2_r, b2_r, o_r):
        z = jnp.mean(x_r[...], axis=0, keepdims=True)
        h = jnp.maximum(
            jnp.dot(z, w1_r[...], preferred_element_type=jnp.float32)
            + b1_r[...], 0.0)
        o_r[...] = jnp.dot(h, w2_r[...],
                           preferred_element_type=jnp.float32) + b2_r[...]

    return pl.pallas_call(
        body, out_shape=jax.ShapeDtypeStruct((1, f_out), jnp.float32))(
            x, w1t, b1.reshape(1, -1), w2t, b2.reshape(1, -1))


def kernel(x, edge_index, edge_attr, params):
    src = edge_index[0]
    dst = edge_index[1]
    zrows = jnp.zeros((_RPT, _D), jnp.float32)

    for pa, pb in zip(params["atom_layers"], params["bond_layers"]):
        wkqvs = jnp.concatenate(
            [pa["key"]["W"], pa["query"]["W"], pa["value"]["W"],
             pa["skip"]["W"]], axis=0).T
        bkqvs = jnp.concatenate(
            [jnp.zeros((_D,), jnp.float32), jnp.zeros((_D,), jnp.float32),
             pa["value"]["b"], pa["skip"]["b"]])
        kqvs = _linear(x, wkqvs, bkqvs, br=2000)
        kk, qq, vv = kqvs[:, :_D], kqvs[:, _D:2 * _D], kqvs[:, 2 * _D:3 * _D]
        skip_o = kqvs[:, 3 * _D:]
        e_bias = pa["edge"]["b"] + pa["key"]["b"] + pa["query"]["b"]
        e = _linear(edge_attr, pa["edge"]["W"].T, e_bias, br=2000)
        kd, qs, vs = _sc_gather((kk, qq, vv), (dst, src), (0, 1, 1))
        msg = _edge_msg(kd, qs, e, vs)
        agg = _sc_scatter_add(msg, dst, zrows)
        x = _atom_post(x, skip_o, agg, pa["bn_gamma"], pa["bn_beta"])

        w12 = jnp.concatenate([pb["v1"]["W"], pb["v2"]["W"]], axis=0).T
        a12 = _linear(x, w12, None, br=2000)
        a1, a2 = a12[:, :_D], a12[:, _D:]
        b0 = pb["v0"]["b"] + pb["v1"]["b"] + pb["v2"]["b"]
        e0 = _linear(edge_attr, pb["v0"]["W"].T, b0, br=2000)
        a1g, a2g = _sc_gather((a1, a2), (src, dst), (0, 1))
        s, stats = _bond_sum_stats(e0, a1g, a2g)
        edge_attr = _bond_post(edge_attr, s, stats, pb["bn_gamma"],
                               pb["bn_beta"])

    boa = _atom_head(x, params["atom_mlp"]["l1"]["W"].T,
                     params["atom_mlp"]["l1"]["b"],
                     params["atom_mlp"]["l2"]["W"].T,
                     params["atom_mlp"]["l2"]["b"])
    hb = _linear(edge_attr, params["bond_mlp"]["l1"]["W"].T,
                 params["bond_mlp"]["l1"]["b"], act="relu", br=2000)
    bonds = _linear(hb, params["bond_mlp"]["l2"]["W"].T,
                    params["bond_mlp"]["l2"]["b"], br=2000)
    return boa.reshape(-1, 8, 100), bonds


# speedup vs baseline: 3.1837x; 1.1914x over previous
import functools

import jax
import jax.numpy as jnp
from jax import lax
from jax.experimental import pallas as pl
from jax.experimental.pallas import tpu as pltpu
from jax.experimental.pallas import tpu_sc as plsc

_N = 10000
_E = 160000
_D = 128
_NC = 2
_NS = 16
_NW = _NC * _NS
_RPW = _E // _NW
_CH = 128
_NFULL = _RPW // _CH
_TAIL = _RPW - _NFULL * _CH
_NPAD = 10240
_RPT = _NPAD // _NS


def _sc_mesh():
    return plsc.VectorSubcoreMesh(core_axis_name="c", subcore_axis_name="s",
                                  num_cores=_NC, num_subcores=_NS)


_NBUF = 3
_SNBUF = 2
_CHUNKS = [(c * _CH, _CH) for c in range(_NFULL)] + [(_NFULL * _CH, _TAIL)]


def _sc_gather(tables, idxs, idx_of):
    n_t = len(tables)
    n_i = len(idxs)
    scratch = (
        [pltpu.VMEM((_RPW,), jnp.int32) for _ in range(n_i)]
        + [pltpu.VMEM((_CH, _D), jnp.float32) for _ in range(_NBUF)]
        + [pltpu.SemaphoreType.DMA, pltpu.SemaphoreType.DMA]
    )
    out_type = tuple(
        jax.ShapeDtypeStruct((_E, _D), jnp.float32) for _ in range(n_t))

    def body(*refs):
        tab = refs[:n_t]
        idx = refs[n_t:n_t + n_i]
        outs = refs[n_t + n_i:n_t + n_i + n_t]
        s = n_t + n_i + n_t
        ib = refs[s:s + n_i]; s += n_i
        rb = refs[s:s + _NBUF]; s += _NBUF
        sem_g, sem_w = refs[s], refs[s + 1]
        wid = lax.axis_index("s") * _NC + lax.axis_index("c")
        base = wid * _RPW
        for j in range(n_i):
            pltpu.sync_copy(idx[j].at[pl.ds(base, _RPW)], ib[j])

        jobs = [(t, off, sz) for t in range(n_t) for (off, sz) in _CHUNKS]
        pend_w = [None] * _NBUF
        prev = None

        def buf(b, sz):
            return rb[b] if sz == _CH else rb[b].at[pl.ds(0, sz)]

        def flush_prev():
            pb, pt, poff, psz, pg = prev
            pg.wait()
            pend_w[pb] = pltpu.async_copy(
                buf(pb, psz), outs[pt].at[pl.ds(base + poff, psz)], sem_w)

        for c, (t, off, sz) in enumerate(jobs):
            b = c % _NBUF
            if pend_w[b] is not None:
                pend_w[b].wait()
                pend_w[b] = None
            g = pltpu.async_copy(
                tab[t].at[ib[idx_of[t]].at[pl.ds(off, sz)]], buf(b, sz), sem_g)
            if prev is not None:
                flush_prev()
            prev = (b, t, off, sz, g)
        flush_prev()
        for d in pend_w:
            if d is not None:
                d.wait()

    f = pl.kernel(body, out_type=out_type, mesh=_sc_mesh(),
                  scratch_types=scratch)
    return f(*tables, *idxs)


def _sc_scatter_add(msg, dst, zrows):
    scratch = (
        [pltpu.VMEM((_CH,), jnp.int32) for _ in range(_SNBUF)]
        + [pltpu.VMEM((_CH, _D), jnp.float32) for _ in range(_SNBUF)]
        + [pltpu.VMEM((_TAIL,), jnp.int32),
           pltpu.VMEM((_TAIL, _D), jnp.float32),
           pltpu.VMEM_SHARED((_NPAD, _D), jnp.float32),
           pltpu.SemaphoreType.DMA, pltpu.SemaphoreType.DMA]
    )

    def body(msg_hbm, dst_hbm, z_hbm, out_hbm, *rest):
        ib = rest[:_SNBUF]
        rb = rest[_SNBUF:2 * _SNBUF]
        ib_t, rb_t, acc, sem_l, sem_s = rest[2 * _SNBUF:]
        cid = lax.axis_index("c")
        sid = lax.axis_index("s")
        wid = sid * _NC + cid
        base = wid * _RPW
        pltpu.sync_copy(z_hbm, acc.at[pl.ds(sid * _RPT, _RPT)])
        plsc.subcore_barrier()

        pend_s = [None] * (_SNBUF + 1)
        prev = None

        def flush_prev():
            slot, pib, prb, li, lm = prev
            li.wait()
            lm.wait()
            pend_s[slot] = pltpu.async_copy(prb, acc.at[pib], sem_s, add=True)

        for c, (off, sz) in enumerate(_CHUNKS):
            if sz == _CH:
                slot = c % _SNBUF
                cib, crb = ib[slot], rb[slot]
            else:
                slot = _SNBUF
                cib, crb = ib_t, rb_t
            if pend_s[slot] is not None:
                pend_s[slot].wait()
                pend_s[slot] = None
            li = pltpu.async_copy(dst_hbm.at[pl.ds(base + off, sz)], cib, sem_l)
            lm = pltpu.async_copy(msg_hbm.at[pl.ds(base + off, sz)], crb, sem_l)
            if prev is not None:
                flush_prev()
            prev = (slot, cib, crb, li, lm)
        flush_prev()
        for d in pend_s:
            if d is not None:
                d.wait()
        plsc.subcore_barrier()
        pltpu.sync_copy(acc.at[pl.ds(sid * _RPT, _RPT)],
                        out_hbm.at[cid, pl.ds(sid * _RPT, _RPT)])

    f = pl.kernel(body,
                  out_type=jax.ShapeDtypeStruct((_NC, _NPAD, _D), jnp.float32),
                  mesh=_sc_mesh(), scratch_types=scratch)
    return f(msg, dst, zrows)


def _linear(x, wt, b=None, act=None, br=2000):
    r, d = x.shape
    f_out = wt.shape[1]
    assert r % br == 0
    grid = (r // br,)
    in_specs = [pl.BlockSpec((br, d), lambda i: (i, 0)),
                pl.BlockSpec((d, f_out), lambda i: (0, 0))]
    args = [x, wt]
    if b is not None:
        in_specs.append(pl.BlockSpec((1, f_out), lambda i: (0, 0)))
        args.append(b.reshape(1, f_out))

    def body(x_ref, w_ref, *rest):
        if b is not None:
            b_ref, o_ref = rest
        else:
            (o_ref,) = rest
        y = jnp.dot(x_ref[...], w_ref[...], preferred_element_type=jnp.float32)
        if b is not None:
            y = y + b_ref[...]
        if act == "relu":
            y = jnp.maximum(y, 0.0)
        o_ref[...] = y

    return pl.pallas_call(
        body, grid=grid, in_specs=in_specs,
        out_specs=pl.BlockSpec((br, f_out), lambda i: (i, 0)),
        out_shape=jax.ShapeDtypeStruct((r, f_out), jnp.float32))(*args)


def _edge_msg(ea, wt, b, kd, qs, vs, br=2000):
    grid = (_E // br,)
    spec = pl.BlockSpec((br, _D), lambda i: (i, 0))
    wspec = pl.BlockSpec((_D, _D), lambda i: (0, 0))
    bspec = pl.BlockSpec((1, _D), lambda i: (0, 0))

    def body(ea_r, w_r, b_r, kd_r, qs_r, vs_r, o_r):
        e = jnp.dot(ea_r[...], w_r[...],
                    preferred_element_type=jnp.float32) + b_r[...]
        o_r[...] = jax.nn.sigmoid(kd_r[...] + qs_r[...] + e) * vs_r[...]

    return pl.pallas_call(
        body, grid=grid, in_specs=[spec, wspec, bspec, spec, spec, spec],
        out_specs=spec,
        out_shape=jax.ShapeDtypeStruct((_E, _D), jnp.float32))(
            ea, wt, b.reshape(1, _D), kd, qs, vs)


def _atom_post(x, skip_o, agg, gamma, beta):
    def body(x_r, s_r, a_r, g_r, b_r, o_r):
        h = s_r[...] + a_r[0, :_N] + a_r[1, :_N]
        mu = jnp.mean(h, axis=0, keepdims=True)
        var = jnp.mean((h - mu) ** 2, axis=0, keepdims=True)
        hn = g_r[...] * (h - mu) * lax.rsqrt(var + 1e-5) + b_r[...]
        o_r[...] = x_r[...] + jnp.maximum(hn, 0.0)

    return pl.pallas_call(
        body, out_shape=jax.ShapeDtypeStruct((_N, _D), jnp.float32))(
            x, skip_o, agg, gamma.reshape(1, _D), beta.reshape(1, _D))


def _bond_sum_stats(ea, wt, b, a1g, a2g, br=2000):
    grid = (_E // br,)
    spec = pl.BlockSpec((br, _D), lambda i: (i, 0))
    wspec = pl.BlockSpec((_D, _D), lambda i: (0, 0))
    bspec = pl.BlockSpec((1, _D), lambda i: (0, 0))

    def body(ea_r, w_r, b_r, a1_r, a2_r, s_r, st_r, sacc, qacc):
        i = pl.program_id(0)
        v = (jnp.dot(ea_r[...], w_r[...], preferred_element_type=jnp.float32)
             + b_r[...] + a1_r[...] + a2_r[...])
        s_r[...] = v

        @pl.when(i == 0)
        def _():
            sacc[...] = jnp.zeros_like(sacc)
            qacc[...] = jnp.zeros_like(qacc)

        sacc[...] += jnp.sum(v, axis=0).reshape(1, _D)
        qacc[...] += jnp.sum(v * v, axis=0).reshape(1, _D)

        @pl.when(i == grid[0] - 1)
        def _():
            st_r[0:1, :] = sacc[...]
            st_r[1:2, :] = qacc[...]

    return pl.pallas_call(
        body, grid=grid, in_specs=[spec, wspec, bspec, spec, spec],
        out_specs=(spec, pl.BlockSpec((2, _D), lambda i: (0, 0))),
        out_shape=(jax.ShapeDtypeStruct((_E, _D), jnp.float32),
                   jax.ShapeDtypeStruct((2, _D), jnp.float32)),
        scratch_shapes=[pltpu.VMEM((1, _D), jnp.float32),
                        pltpu.VMEM((1, _D), jnp.float32)])(
            ea, wt, b.reshape(1, _D), a1g, a2g)


def _bond_post(ea, s, stats, gamma, beta, br=2000):
    grid = (_E // br,)
    spec = pl.BlockSpec((br, _D), lambda i: (i, 0))
    one = pl.BlockSpec((1, _D), lambda i: (0, 0))

    def body(ea_r, s_r, st_r, g_r, b_r, o_r):
        mu = st_r[0:1, :] * (1.0 / _E)
        var = st_r[1:2, :] * (1.0 / _E) - mu * mu
        hn = g_r[...] * (s_r[...] - mu) * lax.rsqrt(var + 1e-5) + b_r[...]
        o_r[...] = ea_r[...] + jnp.maximum(hn, 0.0)

    return pl.pallas_call(
        body, grid=grid,
        in_specs=[spec, spec, pl.BlockSpec((2, _D), lambda i: (0, 0)), one, one],
        out_specs=spec,
        out_shape=jax.ShapeDtypeStruct((_E, _D), jnp.float32))(
            ea, s, stats, gamma.reshape(1, _D), beta.reshape(1, _D))


def _bonds_head(ea, w1t, b1, w2t, b2, br=2000):
    grid = (_E // br,)
    f_out = w2t.shape[1]
    spec = pl.BlockSpec((br, _D), lambda i: (i, 0))

    def body(ea_r, w1_r, b1_r, w2_r, b2_r, o_r):
        h = jnp.maximum(
            jnp.dot(ea_r[...], w1_r[...], preferred_element_type=jnp.float32)
            + b1_r[...], 0.0)
        o_r[...] = jnp.dot(h, w2_r[...],
                           preferred_element_type=jnp.float32) + b2_r[...]

    return pl.pallas_call(
        body, grid=grid,
        in_specs=[spec, pl.BlockSpec((_D, _D), lambda i: (0, 0)),
                  pl.BlockSpec((1, _D), lambda i: (0, 0)),
                  pl.BlockSpec((_D, f_out), lambda i: (0, 0)),
                  pl.BlockSpec((1, f_out), lambda i: (0, 0))],
        out_specs=pl.BlockSpec((br, f_out), lambda i: (i, 0)),
        out_shape=jax.ShapeDtypeStruct((_E, f_out), jnp.float32))(
            ea, w1t, b1.reshape(1, _D), w2t, b2.reshape(1, f_out))


def _atom_head(x, w1t, b1, w2t, b2):
    f_out = w2t.shape[1]

    def body(x_r, w1_r, b1_r, w2_r, b2_r, o_r):
        z = jnp.mean(x_r[...], axis=0, keepdims=True)
        h = jnp.maximum(
            jnp.dot(z, w1_r[...], preferred_element_type=jnp.float32)
            + b1_r[...], 0.0)
        o_r[...] = jnp.dot(h, w2_r[...],
                           preferred_element_type=jnp.float32) + b2_r[...]

    return pl.pallas_call(
        body, out_shape=jax.ShapeDtypeStruct((1, f_out), jnp.float32))(
            x, w1t, b1.reshape(1, -1), w2t, b2.reshape(1, -1))


def kernel(x, edge_index, edge_attr, params):
    src = edge_index[0]
    dst = edge_index[1]
    zrows = jnp.zeros((_RPT, _D), jnp.float32)

    for pa, pb in zip(params["atom_layers"], params["bond_layers"]):
        wkqvs = jnp.concatenate(
            [pa["key"]["W"], pa["query"]["W"], pa["value"]["W"],
             pa["skip"]["W"]], axis=0).T
        bkqvs = jnp.concatenate(
            [jnp.zeros((_D,), jnp.float32), jnp.zeros((_D,), jnp.float32),
             pa["value"]["b"], pa["skip"]["b"]])
        kqvs = _linear(x, wkqvs, bkqvs, br=2000)
        kk, qq, vv = kqvs[:, :_D], kqvs[:, _D:2 * _D], kqvs[:, 2 * _D:3 * _D]
        skip_o = kqvs[:, 3 * _D:]
        e_bias = pa["edge"]["b"] + pa["key"]["b"] + pa["query"]["b"]
        kd, qs, vs = _sc_gather((kk, qq, vv), (dst, src), (0, 1, 1))
        msg = _edge_msg(edge_attr, pa["edge"]["W"].T, e_bias, kd, qs, vs)
        agg = _sc_scatter_add(msg, dst, zrows)
        x = _atom_post(x, skip_o, agg, pa["bn_gamma"], pa["bn_beta"])

        w12 = jnp.concatenate([pb["v1"]["W"], pb["v2"]["W"]], axis=0).T
        a12 = _linear(x, w12, None, br=2000)
        a1, a2 = a12[:, :_D], a12[:, _D:]
        b0 = pb["v0"]["b"] + pb["v1"]["b"] + pb["v2"]["b"]
        a1g, a2g = _sc_gather((a1, a2), (src, dst), (0, 1))
        s, stats = _bond_sum_stats(edge_attr, pb["v0"]["W"].T, b0, a1g, a2g)
        edge_attr = _bond_post(edge_attr, s, stats, pb["bn_gamma"],
                               pb["bn_beta"])

    boa = _atom_head(x, params["atom_mlp"]["l1"]["W"].T,
                     params["atom_mlp"]["l1"]["b"],
                     params["atom_mlp"]["l2"]["W"].T,
                     params["atom_mlp"]["l2"]["b"])
    bonds = _bonds_head(edge_attr, params["bond_mlp"]["l1"]["W"].T,
                        params["bond_mlp"]["l1"]["b"],
                        params["bond_mlp"]["l2"]["W"].T,
                        params["bond_mlp"]["l2"]["b"])
    return boa.reshape(-1, 8, 100), bonds


# speedup vs baseline: 3.3462x; 1.0510x over previous
import functools

import jax
import jax.numpy as jnp
from jax import lax
from jax.experimental import pallas as pl
from jax.experimental.pallas import tpu as pltpu
from jax.experimental.pallas import tpu_sc as plsc

_N = 10000
_E = 160000
_D = 128
_NC = 2
_NS = 16
_NW = _NC * _NS
_RPW = _E // _NW
_CH = 128
_NFULL = _RPW // _CH
_TAIL = _RPW - _NFULL * _CH
_NPAD = 10240
_RPT = _NPAD // _NS


def _sc_mesh():
    return plsc.VectorSubcoreMesh(core_axis_name="c", subcore_axis_name="s",
                                  num_cores=_NC, num_subcores=_NS)


_NBUF = 3
_SNBUF = 2
_CHUNKS = [(c * _CH, _CH) for c in range(_NFULL)] + [(_NFULL * _CH, _TAIL)]


def _sc_gather(tables, idxs, idx_of):
    n_t = len(tables)
    n_i = len(idxs)
    scratch = (
        [pltpu.VMEM((_RPW,), jnp.int32) for _ in range(n_i)]
        + [pltpu.VMEM((_CH, _D), jnp.float32) for _ in range(_NBUF)]
        + [pltpu.SemaphoreType.DMA, pltpu.SemaphoreType.DMA]
    )
    out_type = tuple(
        jax.ShapeDtypeStruct((_E, _D), jnp.float32) for _ in range(n_t))

    def body(*refs):
        tab = refs[:n_t]
        idx = refs[n_t:n_t + n_i]
        outs = refs[n_t + n_i:n_t + n_i + n_t]
        s = n_t + n_i + n_t
        ib = refs[s:s + n_i]; s += n_i
        rb = refs[s:s + _NBUF]; s += _NBUF
        sem_g, sem_w = refs[s], refs[s + 1]
        wid = lax.axis_index("s") * _NC + lax.axis_index("c")
        base = wid * _RPW
        for j in range(n_i):
            pltpu.sync_copy(idx[j].at[pl.ds(base, _RPW)], ib[j])

        jobs = [(t, off, sz) for t in range(n_t) for (off, sz) in _CHUNKS]
        pend_w = [None] * _NBUF
        prev = None

        def buf(b, sz):
            return rb[b] if sz == _CH else rb[b].at[pl.ds(0, sz)]

        def flush_prev():
            pb, pt, poff, psz, pg = prev
            pg.wait()
            pend_w[pb] = pltpu.async_copy(
                buf(pb, psz), outs[pt].at[pl.ds(base + poff, psz)], sem_w)

        for c, (t, off, sz) in enumerate(jobs):
            b = c % _NBUF
            if pend_w[b] is not None:
                pend_w[b].wait()
                pend_w[b] = None
            g = pltpu.async_copy(
                tab[t].at[ib[idx_of[t]].at[pl.ds(off, sz)]], buf(b, sz), sem_g)
            if prev is not None:
                flush_prev()
            prev = (b, t, off, sz, g)
        flush_prev()
        for d in pend_w:
            if d is not None:
                d.wait()

    f = pl.kernel(body, out_type=out_type, mesh=_sc_mesh(),
                  scratch_types=scratch)
    return f(*tables, *idxs)


_MCH = 112
_MNF = _RPW // _MCH
_MTAIL = _RPW - _MNF * _MCH
_MCHUNKS = [(c * _MCH, _MCH) for c in range(_MNF)] + [(_MNF * _MCH, _MTAIL)]


def _sc_atom_msg(kk, qq, vv, e, dst, src):
    scratch = (
        [pltpu.VMEM((_RPW,), jnp.int32) for _ in range(2)]
        + [pltpu.VMEM((_MCH, _D), jnp.float32) for _ in range(8)]
        + [pltpu.SemaphoreType.DMA for _ in range(4)]
    )

    def body(kk_h, qq_h, vv_h, e_h, dst_h, src_h, out_h, *rest):
        ibd, ibs = rest[0], rest[1]
        bk = rest[2:4]
        bq = rest[4:6]
        bv = rest[6:8]
        be = rest[8:10]
        sem_g = rest[10:12]
        sem_w = rest[12:14]
        wid = lax.axis_index("s") * _NC + lax.axis_index("c")
        base = wid * _RPW
        pltpu.sync_copy(dst_h.at[pl.ds(base, _RPW)], ibd)
        pltpu.sync_copy(src_h.at[pl.ds(base, _RPW)], ibs)

        def sl(ref, sz):
            return ref if sz == _MCH else ref.at[pl.ds(0, sz)]

        pend_w = [None, None]
        prev = None

        def compute_and_flush():
            b, off, sz, descs = prev
            for d_ in descs:
                d_.wait()

            def row(r, carry):
                for j in range(_D // 16):
                    s_ = pl.ds(j * 16, 16)
                    xv = bk[b][r, s_] + bq[b][r, s_] + be[b][r, s_]
                    eta = 1.0 / (1.0 + jnp.exp(-xv))
                    bv[b][r, s_] = eta * bv[b][r, s_]
                return carry

            lax.fori_loop(0, sz, row, 0)
            pend_w[b] = pltpu.async_copy(
                sl(bv[b], sz), out_h.at[pl.ds(base + off, sz)], sem_w[b])

        for c, (off, sz) in enumerate(_MCHUNKS):
            b = c % 2
            if pend_w[b] is not None:
                pend_w[b].wait()
                pend_w[b] = None
            ds_ = ibd.at[pl.ds(off, sz)]
            ss_ = ibs.at[pl.ds(off, sz)]
            descs = [
                pltpu.async_copy(kk_h.at[ds_], sl(bk[b], sz), sem_g[b]),
                pltpu.async_copy(qq_h.at[ss_], sl(bq[b], sz), sem_g[b]),
                pltpu.async_copy(vv_h.at[ss_], sl(bv[b], sz), sem_g[b]),
                pltpu.async_copy(e_h.at[pl.ds(base + off, sz)],
                                 sl(be[b], sz), sem_g[b]),
            ]
            if prev is not None:
                compute_and_flush()
            prev = (b, off, sz, descs)
        compute_and_flush()
        for d_ in pend_w:
            if d_ is not None:
                d_.wait()

    f = pl.kernel(body, out_type=jax.ShapeDtypeStruct((_E, _D), jnp.float32),
                  mesh=_sc_mesh(), scratch_types=scratch)
    return f(kk, qq, vv, e, dst, src)


def _sc_scatter_add(msg, dst, zrows):
    scratch = (
        [pltpu.VMEM((_CH,), jnp.int32) for _ in range(_SNBUF)]
        + [pltpu.VMEM((_CH, _D), jnp.float32) for _ in range(_SNBUF)]
        + [pltpu.VMEM((_TAIL,), jnp.int32),
           pltpu.VMEM((_TAIL, _D), jnp.float32),
           pltpu.VMEM_SHARED((_NPAD, _D), jnp.float32),
           pltpu.SemaphoreType.DMA, pltpu.SemaphoreType.DMA]
    )

    def body(msg_hbm, dst_hbm, z_hbm, out_hbm, *rest):
        ib = rest[:_SNBUF]
        rb = rest[_SNBUF:2 * _SNBUF]
        ib_t, rb_t, acc, sem_l, sem_s = rest[2 * _SNBUF:]
        cid = lax.axis_index("c")
        sid = lax.axis_index("s")
        wid = sid * _NC + cid
        base = wid * _RPW
        pltpu.sync_copy(z_hbm, acc.at[pl.ds(sid * _RPT, _RPT)])
        plsc.subcore_barrier()

        pend_s = [None] * (_SNBUF + 1)
        prev = None

        def flush_prev():
            slot, pib, prb, li, lm = prev
            li.wait()
            lm.wait()
            pend_s[slot] = pltpu.async_copy(prb, acc.at[pib], sem_s, add=True)

        for c, (off, sz) in enumerate(_CHUNKS):
            if sz == _CH:
                slot = c % _SNBUF
                cib, crb = ib[slot], rb[slot]
            else:
                slot = _SNBUF
                cib, crb = ib_t, rb_t
            if pend_s[slot] is not None:
                pend_s[slot].wait()
                pend_s[slot] = None
            li = pltpu.async_copy(dst_hbm.at[pl.ds(base + off, sz)], cib, sem_l)
            lm = pltpu.async_copy(msg_hbm.at[pl.ds(base + off, sz)], crb, sem_l)
            if prev is not None:
                flush_prev()
            prev = (slot, cib, crb, li, lm)
        flush_prev()
        for d in pend_s:
            if d is not None:
                d.wait()
        plsc.subcore_barrier()
        pltpu.sync_copy(acc.at[pl.ds(sid * _RPT, _RPT)],
                        out_hbm.at[cid, pl.ds(sid * _RPT, _RPT)])

    f = pl.kernel(body,
                  out_type=jax.ShapeDtypeStruct((_NC, _NPAD, _D), jnp.float32),
                  mesh=_sc_mesh(), scratch_types=scratch)
    return f(msg, dst, zrows)


def _linear(x, wt, b=None, act=None, br=2000):
    r, d = x.shape
    f_out = wt.shape[1]
    assert r % br == 0
    grid = (r // br,)
    in_specs = [pl.BlockSpec((br, d), lambda i: (i, 0)),
                pl.BlockSpec((d, f_out), lambda i: (0, 0))]
    args = [x, wt]
    if b is not None:
        in_specs.append(pl.BlockSpec((1, f_out), lambda i: (0, 0)))
        args.append(b.reshape(1, f_out))

    def body(x_ref, w_ref, *rest):
        if b is not None:
            b_ref, o_ref = rest
        else:
            (o_ref,) = rest
        y = jnp.dot(x_ref[...], w_ref[...], preferred_element_type=jnp.float32)
        if b is not None:
            y = y + b_ref[...]
        if act == "relu":
            y = jnp.maximum(y, 0.0)
        o_ref[...] = y

    return pl.pallas_call(
        body, grid=grid, in_specs=in_specs,
        out_specs=pl.BlockSpec((br, f_out), lambda i: (i, 0)),
        out_shape=jax.ShapeDtypeStruct((r, f_out), jnp.float32))(*args)


def _atom_post(x, skip_o, agg, gamma, beta):
    def body(x_r, s_r, a_r, g_r, b_r, o_r):
        h = s_r[...] + a_r[0, :_N] + a_r[1, :_N]
        mu = jnp.mean(h, axis=0, keepdims=True)
        var = jnp.mean((h - mu) ** 2, axis=0, keepdims=True)
        hn = g_r[...] * (h - mu) * lax.rsqrt(var + 1e-5) + b_r[...]
        o_r[...] = x_r[...] + jnp.maximum(hn, 0.0)

    return pl.pallas_call(
        body, out_shape=jax.ShapeDtypeStruct((_N, _D), jnp.float32))(
            x, skip_o, agg, gamma.reshape(1, _D), beta.reshape(1, _D))


def _bond_sum_stats(ea, wt, b, a1g, a2g, br=2000):
    grid = (_E // br,)
    spec = pl.BlockSpec((br, _D), lambda i: (i, 0))
    wspec = pl.BlockSpec((_D, _D), lambda i: (0, 0))
    bspec = pl.BlockSpec((1, _D), lambda i: (0, 0))

    def body(ea_r, w_r, b_r, a1_r, a2_r, s_r, st_r, sacc, qacc):
        i = pl.program_id(0)
        v = (jnp.dot(ea_r[...], w_r[...], preferred_element_type=jnp.float32)
             + b_r[...] + a1_r[...] + a2_r[...])
        s_r[...] = v

        @pl.when(i == 0)
        def _():
            sacc[...] = jnp.zeros_like(sacc)
            qacc[...] = jnp.zeros_like(qacc)

        sacc[...] += jnp.sum(v, axis=0).reshape(1, _D)
        qacc[...] += jnp.sum(v * v, axis=0).reshape(1, _D)

        @pl.when(i == grid[0] - 1)
        def _():
            st_r[0:1, :] = sacc[...]
            st_r[1:2, :] = qacc[...]

    return pl.pallas_call(
        body, grid=grid, in_specs=[spec, wspec, bspec, spec, spec],
        out_specs=(spec, pl.BlockSpec((2, _D), lambda i: (0, 0))),
        out_shape=(jax.ShapeDtypeStruct((_E, _D), jnp.float32),
                   jax.ShapeDtypeStruct((2, _D), jnp.float32)),
        scratch_shapes=[pltpu.VMEM((1, _D), jnp.float32),
                        pltpu.VMEM((1, _D), jnp.float32)])(
            ea, wt, b.reshape(1, _D), a1g, a2g)


def _bond_post(ea, s, stats, gamma, beta, br=2000):
    grid = (_E // br,)
    spec = pl.BlockSpec((br, _D), lambda i: (i, 0))
    one = pl.BlockSpec((1, _D), lambda i: (0, 0))

    def body(ea_r, s_r, st_r, g_r, b_r, o_r):
        mu = st_r[0:1, :] * (1.0 / _E)
        var = st_r[1:2, :] * (1.0 / _E) - mu * mu
        hn = g_r[...] * (s_r[...] - mu) * lax.rsqrt(var + 1e-5) + b_r[...]
        o_r[...] = ea_r[...] + jnp.maximum(hn, 0.0)

    return pl.pallas_call(
        body, grid=grid,
        in_specs=[spec, spec, pl.BlockSpec((2, _D), lambda i: (0, 0)), one, one],
        out_specs=spec,
        out_shape=jax.ShapeDtypeStruct((_E, _D), jnp.float32))(
            ea, s, stats, gamma.reshape(1, _D), beta.reshape(1, _D))


def _bonds_head(ea, w1t, b1, w2t, b2, br=2000):
    grid = (_E // br,)
    f_out = w2t.shape[1]
    spec = pl.BlockSpec((br, _D), lambda i: (i, 0))

    def body(ea_r, w1_r, b1_r, w2_r, b2_r, o_r):
        h = jnp.maximum(
            jnp.dot(ea_r[...], w1_r[...], preferred_element_type=jnp.float32)
            + b1_r[...], 0.0)
        o_r[...] = jnp.dot(h, w2_r[...],
                           preferred_element_type=jnp.float32) + b2_r[...]

    return pl.pallas_call(
        body, grid=grid,
        in_specs=[spec, pl.BlockSpec((_D, _D), lambda i: (0, 0)),
                  pl.BlockSpec((1, _D), lambda i: (0, 0)),
                  pl.BlockSpec((_D, f_out), lambda i: (0, 0)),
                  pl.BlockSpec((1, f_out), lambda i: (0, 0))],
        out_specs=pl.BlockSpec((br, f_out), lambda i: (i, 0)),
        out_shape=jax.ShapeDtypeStruct((_E, f_out), jnp.float32))(
            ea, w1t, b1.reshape(1, _D), w2t, b2.reshape(1, f_out))


def _atom_head(x, w1t, b1, w2t, b2):
    f_out = w2t.shape[1]

    def body(x_r, w1_r, b1_r, w2_r, b2_r, o_r):
        z = jnp.mean(x_r[...], axis=0, keepdims=True)
        h = jnp.maximum(
            jnp.dot(z, w1_r[...], preferred_element_type=jnp.float32)
            + b1_r[...], 0.0)
        o_r[...] = jnp.dot(h, w2_r[...],
                           preferred_element_type=jnp.float32) + b2_r[...]

    return pl.pallas_call(
        body, out_shape=jax.ShapeDtypeStruct((1, f_out), jnp.float32))(
            x, w1t, b1.reshape(1, -1), w2t, b2.reshape(1, -1))


def kernel(x, edge_index, edge_attr, params):
    src = edge_index[0]
    dst = edge_index[1]
    zrows = jnp.zeros((_RPT, _D), jnp.float32)

    for pa, pb in zip(params["atom_layers"], params["bond_layers"]):
        wkqvs = jnp.concatenate(
            [pa["key"]["W"], pa["query"]["W"], pa["value"]["W"],
             pa["skip"]["W"]], axis=0).T
        bkqvs = jnp.concatenate(
            [jnp.zeros((_D,), jnp.float32), jnp.zeros((_D,), jnp.float32),
             pa["value"]["b"], pa["skip"]["b"]])
        kqvs = _linear(x, wkqvs, bkqvs, br=2000)
        kk, qq, vv = kqvs[:, :_D], kqvs[:, _D:2 * _D], kqvs[:, 2 * _D:3 * _D]
        skip_o = kqvs[:, 3 * _D:]
        e_bias = pa["edge"]["b"] + pa["key"]["b"] + pa["query"]["b"]
        e = _linear(edge_attr, pa["edge"]["W"].T, e_bias, br=2000)
        msg = _sc_atom_msg(kk, qq, vv, e, dst, src)
        agg = _sc_scatter_add(msg, dst, zrows)
        x = _atom_post(x, skip_o, agg, pa["bn_gamma"], pa["bn_beta"])

        w12 = jnp.concatenate([pb["v1"]["W"], pb["v2"]["W"]], axis=0).T
        a12 = _linear(x, w12, None, br=2000)
        a1, a2 = a12[:, :_D], a12[:, _D:]
        b0 = pb["v0"]["b"] + pb["v1"]["b"] + pb["v2"]["b"]
        a1g, a2g = _sc_gather((a1, a2), (src, dst), (0, 1))
        s, stats = _bond_sum_stats(edge_attr, pb["v0"]["W"].T, b0, a1g, a2g)
        edge_attr = _bond_post(edge_attr, s, stats, pb["bn_gamma"],
                               pb["bn_beta"])

    boa = _atom_head(x, params["atom_mlp"]["l1"]["W"].T,
                     params["atom_mlp"]["l1"]["b"],
                     params["atom_mlp"]["l2"]["W"].T,
                     params["atom_mlp"]["l2"]["b"])
    bonds = _bonds_head(edge_attr, params["bond_mlp"]["l1"]["W"].T,
                        params["bond_mlp"]["l1"]["b"],
                        params["bond_mlp"]["l2"]["W"].T,
                        params["bond_mlp"]["l2"]["b"])
    return boa.reshape(-1, 8, 100), bonds


# speedup vs baseline: 3.3505x; 1.0013x over previous
import functools

import jax
import jax.numpy as jnp
from jax import lax
from jax.experimental import pallas as pl
from jax.experimental.pallas import tpu as pltpu
from jax.experimental.pallas import tpu_sc as plsc

_N = 10000
_E = 160000
_D = 128
_NC = 2
_NS = 16
_NW = _NC * _NS
_RPW = _E // _NW
_CH = 128
_NFULL = _RPW // _CH
_TAIL = _RPW - _NFULL * _CH
_NPAD = 10240
_RPT = _NPAD // _NS


def _sc_mesh():
    return plsc.VectorSubcoreMesh(core_axis_name="c", subcore_axis_name="s",
                                  num_cores=_NC, num_subcores=_NS)


_NBUF = 3
_SNBUF = 2
_CHUNKS = [(c * _CH, _CH) for c in range(_NFULL)] + [(_NFULL * _CH, _TAIL)]


def _sc_gather(tables, idxs, idx_of):
    n_t = len(tables)
    n_i = len(idxs)
    width = tables[0].shape[1]
    dt = tables[0].dtype
    scratch = (
        [pltpu.VMEM((_RPW,), jnp.int32) for _ in range(n_i)]
        + [pltpu.VMEM((_CH, width), dt) for _ in range(_NBUF)]
        + [pltpu.SemaphoreType.DMA, pltpu.SemaphoreType.DMA]
    )
    out_type = tuple(
        jax.ShapeDtypeStruct((_E, width), dt) for _ in range(n_t))

    def body(*refs):
        tab = refs[:n_t]
        idx = refs[n_t:n_t + n_i]
        outs = refs[n_t + n_i:n_t + n_i + n_t]
        s = n_t + n_i + n_t
        ib = refs[s:s + n_i]; s += n_i
        rb = refs[s:s + _NBUF]; s += _NBUF
        sem_g, sem_w = refs[s], refs[s + 1]
        wid = lax.axis_index("s") * _NC + lax.axis_index("c")
        base = wid * _RPW
        for j in range(n_i):
            pltpu.sync_copy(idx[j].at[pl.ds(base, _RPW)], ib[j])

        jobs = [(t, off, sz) for t in range(n_t) for (off, sz) in _CHUNKS]
        pend_w = [None] * _NBUF
        prev = None

        def buf(b, sz):
            return rb[b] if sz == _CH else rb[b].at[pl.ds(0, sz)]

        def flush_prev():
            pb, pt, poff, psz, pg = prev
            pg.wait()
            pend_w[pb] = pltpu.async_copy(
                buf(pb, psz), outs[pt].at[pl.ds(base + poff, psz)], sem_w)

        for c, (t, off, sz) in enumerate(jobs):
            b = c % _NBUF
            if pend_w[b] is not None:
                pend_w[b].wait()
                pend_w[b] = None
            g = pltpu.async_copy(
                tab[t].at[ib[idx_of[t]].at[pl.ds(off, sz)]], buf(b, sz), sem_g)
            if prev is not None:
                flush_prev()
            prev = (b, t, off, sz, g)
        flush_prev()
        for d in pend_w:
            if d is not None:
                d.wait()

    f = pl.kernel(body, out_type=out_type, mesh=_sc_mesh(),
                  scratch_types=scratch)
    return f(*tables, *idxs)


def _sc_atom_msg(kk, qv, e, dst, src):
    scratch = (
        [pltpu.VMEM((_RPW,), jnp.int32) for _ in range(2)]
        + [pltpu.VMEM((_CH, _D), jnp.float32) for _ in range(2)]
        + [pltpu.VMEM((_CH, _D), jnp.int32) for _ in range(2)]
        + [pltpu.VMEM((_CH, _D), jnp.float32) for _ in range(2)]
        + [pltpu.SemaphoreType.DMA for _ in range(4)]
    )

    def body(kk_h, qv_h, e_h, dst_h, src_h, out_h, *rest):
        ibd, ibs = rest[0], rest[1]
        bk = rest[2:4]
        bqv = rest[4:6]
        be = rest[6:8]
        sem_g = rest[8:10]
        sem_w = rest[10:12]
        wid = lax.axis_index("s") * _NC + lax.axis_index("c")
        base = wid * _RPW
        pltpu.sync_copy(dst_h.at[pl.ds(base, _RPW)], ibd)
        pltpu.sync_copy(src_h.at[pl.ds(base, _RPW)], ibs)

        def sl(ref, sz):
            return ref if sz == _CH else ref.at[pl.ds(0, sz)]

        pend_w = [None, None]
        prev = None

        def compute_and_flush():
            b, off, sz, descs = prev
            for d_ in descs:
                d_.wait()

            def row(r, carry):
                for j in range(_D // 16):
                    s_ = pl.ds(j * 16, 16)
                    x_ = bqv[b][r, s_]
                    qf = ((x_ << 16) >> 16).astype(jnp.float32) * (1.0 / 256.0)
                    vf = (x_ >> 16).astype(jnp.float32) * (1.0 / 256.0)
                    xv = bk[b][r, s_] + qf + be[b][r, s_]
                    eta = 1.0 / (1.0 + jnp.exp(-xv))
                    be[b][r, s_] = eta * vf
                return carry

            lax.fori_loop(0, sz, row, 0)
            pend_w[b] = pltpu.async_copy(
                sl(be[b], sz), out_h.at[pl.ds(base + off, sz)], sem_w[b])

        for c, (off, sz) in enumerate(_CHUNKS):
            b = c % 2
            if pend_w[b] is not None:
                pend_w[b].wait()
                pend_w[b] = None
            ds_ = ibd.at[pl.ds(off, sz)]
            ss_ = ibs.at[pl.ds(off, sz)]
            descs = [
                pltpu.async_copy(kk_h.at[ds_], sl(bk[b], sz), sem_g[b]),
                pltpu.async_copy(qv_h.at[ss_], sl(bqv[b], sz), sem_g[b]),
                pltpu.async_copy(e_h.at[pl.ds(base + off, sz)],
                                 sl(be[b], sz), sem_g[b]),
            ]
            if prev is not None:
                compute_and_flush()
            prev = (b, off, sz, descs)
        compute_and_flush()
        for d_ in pend_w:
            if d_ is not None:
                d_.wait()

    f = pl.kernel(body, out_type=jax.ShapeDtypeStruct((_E, _D), jnp.float32),
                  mesh=_sc_mesh(), scratch_types=scratch)
    return f(kk, qv, e, dst, src)


def _sc_scatter_add(msg, dst, zrows):
    scratch = (
        [pltpu.VMEM((_CH,), jnp.int32) for _ in range(_SNBUF)]
        + [pltpu.VMEM((_CH, _D), jnp.float32) for _ in range(_SNBUF)]
        + [pltpu.VMEM((_TAIL,), jnp.int32),
           pltpu.VMEM((_TAIL, _D), jnp.float32),
           pltpu.VMEM_SHARED((_NPAD, _D), jnp.float32),
           pltpu.SemaphoreType.DMA, pltpu.SemaphoreType.DMA]
    )

    def body(msg_hbm, dst_hbm, z_hbm, out_hbm, *rest):
        ib = rest[:_SNBUF]
        rb = rest[_SNBUF:2 * _SNBUF]
        ib_t, rb_t, acc, sem_l, sem_s = rest[2 * _SNBUF:]
        cid = lax.axis_index("c")
        sid = lax.axis_index("s")
        wid = sid * _NC + cid
        base = wid * _RPW
        pltpu.sync_copy(z_hbm, acc.at[pl.ds(sid * _RPT, _RPT)])
        plsc.subcore_barrier()

        pend_s = [None] * (_SNBUF + 1)
        prev = None

        def flush_prev():
            slot, pib, prb, li, lm = prev
            li.wait()
            lm.wait()
            pend_s[slot] = pltpu.async_copy(prb, acc.at[pib], sem_s, add=True)

        for c, (off, sz) in enumerate(_CHUNKS):
            if sz == _CH:
                slot = c % _SNBUF
                cib, crb = ib[slot], rb[slot]
            else:
                slot = _SNBUF
                cib, crb = ib_t, rb_t
            if pend_s[slot] is not None:
                pend_s[slot].wait()
                pend_s[slot] = None
            li = pltpu.async_copy(dst_hbm.at[pl.ds(base + off, sz)], cib, sem_l)
            lm = pltpu.async_copy(msg_hbm.at[pl.ds(base + off, sz)], crb, sem_l)
            if prev is not None:
                flush_prev()
            prev = (slot, cib, crb, li, lm)
        flush_prev()
        for d in pend_s:
            if d is not None:
                d.wait()
        plsc.subcore_barrier()
        pltpu.sync_copy(acc.at[pl.ds(sid * _RPT, _RPT)],
                        out_hbm.at[cid, pl.ds(sid * _RPT, _RPT)])

    f = pl.kernel(body,
                  out_type=jax.ShapeDtypeStruct((_NC, _NPAD, _D), jnp.float32),
                  mesh=_sc_mesh(), scratch_types=scratch)
    return f(msg, dst, zrows)


def _linear(x, wt, b=None, act=None, br=2000, out_dtype=jnp.float32):
    r, d = x.shape
    f_out = wt.shape[1]
    assert r % br == 0
    grid = (r // br,)
    in_specs = [pl.BlockSpec((br, d), lambda i: (i, 0)),
                pl.BlockSpec((d, f_out), lambda i: (0, 0))]
    args = [x, wt]
    if b is not None:
        in_specs.append(pl.BlockSpec((1, f_out), lambda i: (0, 0)))
        args.append(b.reshape(1, f_out))

    def body(x_ref, w_ref, *rest):
        if b is not None:
            b_ref, o_ref = rest
        else:
            (o_ref,) = rest
        y = jnp.dot(x_ref[...], w_ref[...], preferred_element_type=jnp.float32)
        if b is not None:
            y = y + b_ref[...]
        if act == "relu":
            y = jnp.maximum(y, 0.0)
        o_ref[...] = y.astype(out_dtype)

    return pl.pallas_call(
        body, grid=grid, in_specs=in_specs,
        out_specs=pl.BlockSpec((br, f_out), lambda i: (i, 0)),
        out_shape=jax.ShapeDtypeStruct((r, f_out), out_dtype))(*args)


def _atom_post(x, skip_o, agg, gamma, beta):
    def body(x_r, s_r, a_r, g_r, b_r, o_r):
        h = s_r[...] + a_r[0, :_N] + a_r[1, :_N]
        mu = jnp.mean(h, axis=0, keepdims=True)
        var = jnp.mean((h - mu) ** 2, axis=0, keepdims=True)
        hn = g_r[...] * (h - mu) * lax.rsqrt(var + 1e-5) + b_r[...]
        o_r[...] = x_r[...] + jnp.maximum(hn, 0.0)

    return pl.pallas_call(
        body, out_shape=jax.ShapeDtypeStruct((_N, _D), jnp.float32))(
            x, skip_o, agg, gamma.reshape(1, _D), beta.reshape(1, _D))


def _bond_sum_stats(ea, wt, b, a1g, a2g, br=2000):
    grid = (_E // br,)
    spec = pl.BlockSpec((br, _D), lambda i: (i, 0))
    wspec = pl.BlockSpec((_D, _D), lambda i: (0, 0))
    bspec = pl.BlockSpec((1, _D), lambda i: (0, 0))

    def body(ea_r, w_r, b_r, a1_r, a2_r, s_r, st_r, sacc, qacc):
        i = pl.program_id(0)
        v = (jnp.dot(ea_r[...], w_r[...], preferred_element_type=jnp.float32)
             + b_r[...] + a1_r[...] + a2_r[...])
        s_r[...] = v

        @pl.when(i == 0)
        def _():
            sacc[...] = jnp.zeros_like(sacc)
            qacc[...] = jnp.zeros_like(qacc)

        sacc[...] += jnp.sum(v, axis=0).reshape(1, _D)
        qacc[...] += jnp.sum(v * v, axis=0).reshape(1, _D)

        @pl.when(i == grid[0] - 1)
        def _():
            st_r[0:1, :] = sacc[...]
            st_r[1:2, :] = qacc[...]

    return pl.pallas_call(
        body, grid=grid, in_specs=[spec, wspec, bspec, spec, spec],
        out_specs=(spec, pl.BlockSpec((2, _D), lambda i: (0, 0))),
        out_shape=(jax.ShapeDtypeStruct((_E, _D), jnp.float32),
                   jax.ShapeDtypeStruct((2, _D), jnp.float32)),
        scratch_shapes=[pltpu.VMEM((1, _D), jnp.float32),
                        pltpu.VMEM((1, _D), jnp.float32)])(
            ea, wt, b.reshape(1, _D), a1g, a2g)


def _bond_post(ea, s, stats, gamma, beta, br=2000):
    grid = (_E // br,)
    spec = pl.BlockSpec((br, _D), lambda i: (i, 0))
    one = pl.BlockSpec((1, _D), lambda i: (0, 0))

    def body(ea_r, s_r, st_r, g_r, b_r, o_r):
        mu = st_r[0:1, :] * (1.0 / _E)
        var = st_r[1:2, :] * (1.0 / _E) - mu * mu
        hn = g_r[...] * (s_r[...] - mu) * lax.rsqrt(var + 1e-5) + b_r[...]
        o_r[...] = ea_r[...] + jnp.maximum(hn, 0.0)

    return pl.pallas_call(
        body, grid=grid,
        in_specs=[spec, spec, pl.BlockSpec((2, _D), lambda i: (0, 0)), one, one],
        out_specs=spec,
        out_shape=jax.ShapeDtypeStruct((_E, _D), jnp.float32))(
            ea, s, stats, gamma.reshape(1, _D), beta.reshape(1, _D))


def _atom_proj(x, wt, b, br=2000):
    grid = (_N // br,)

    def body(x_ref, w_ref, b_ref, ks_ref, qv_ref):
        y = jnp.dot(x_ref[...], w_ref[...],
                    preferred_element_type=jnp.float32) + b_ref[...]
        ks_ref[:, :_D] = y[:, :_D]
        ks_ref[:, _D:] = y[:, 3 * _D:]
        qi = jnp.clip(jnp.round(y[:, _D:2 * _D] * 256.0), -32768.0,
                      32767.0).astype(jnp.int32)
        vi = jnp.clip(jnp.round(y[:, 2 * _D:3 * _D] * 256.0), -32768.0,
                      32767.0).astype(jnp.int32)
        qv_ref[...] = (vi << 16) | (qi & 0xFFFF)

    return pl.pallas_call(
        body, grid=grid,
        in_specs=[pl.BlockSpec((br, _D), lambda i: (i, 0)),
                  pl.BlockSpec((_D, 4 * _D), lambda i: (0, 0)),
                  pl.BlockSpec((1, 4 * _D), lambda i: (0, 0))],
        out_specs=(pl.BlockSpec((br, 2 * _D), lambda i: (i, 0)),
                   pl.BlockSpec((br, _D), lambda i: (i, 0))),
        out_shape=(jax.ShapeDtypeStruct((_N, 2 * _D), jnp.float32),
                   jax.ShapeDtypeStruct((_N, _D), jnp.int32)))(
            x, wt, b.reshape(1, 4 * _D))


def _bonds_head(ea, w1t, b1, w2t, b2, br=2000):
    grid = (_E // br,)
    f_out = w2t.shape[1]
    spec = pl.BlockSpec((br, _D), lambda i: (i, 0))

    def body(ea_r, w1_r, b1_r, w2_r, b2_r, o_r):
        h = jnp.maximum(
            jnp.dot(ea_r[...], w1_r[...], preferred_element_type=jnp.float32)
            + b1_r[...], 0.0)
        o_r[...] = jnp.dot(h, w2_r[...],
                           preferred_element_type=jnp.float32) + b2_r[...]

    return pl.pallas_call(
        body, grid=grid,
        in_specs=[spec, pl.BlockSpec((_D, _D), lambda i: (0, 0)),
                  pl.BlockSpec((1, _D), lambda i: (0, 0)),
                  pl.BlockSpec((_D, f_out), lambda i: (0, 0)),
                  pl.BlockSpec((1, f_out), lambda i: (0, 0))],
        out_specs=pl.BlockSpec((br, f_out), lambda i: (i, 0)),
        out_shape=jax.ShapeDtypeStruct((_E, f_out), jnp.float32))(
            ea, w1t, b1.reshape(1, _D), w2t, b2.reshape(1, f_out))


def _atom_head(x, w1t, b1, w2t, b2):
    f_out = w2t.shape[1]

    def body(x_r, w1_r, b1_r, w2_r, b2_r, o_r):
        z = jnp.mean(x_r[...], axis=0, keepdims=True)
        h = jnp.maximum(
            jnp.dot(z, w1_r[...], preferred_element_type=jnp.float32)
            + b1_r[...], 0.0)
        o_r[...] = jnp.dot(h, w2_r[...],
                           preferred_element_type=jnp.float32) + b2_r[...]

    return pl.pallas_call(
        body, out_shape=jax.ShapeDtypeStruct((1, f_out), jnp.float32))(
            x, w1t, b1.reshape(1, -1), w2t, b2.reshape(1, -1))


def kernel(x, edge_index, edge_attr, params):
    src = edge_index[0]
    dst = edge_index[1]
    zrows = jnp.zeros((_RPT, _D), jnp.float32)

    for pa, pb in zip(params["atom_layers"], params["bond_layers"]):
        wkqvs = jnp.concatenate(
            [pa["key"]["W"], pa["query"]["W"], pa["value"]["W"],
             pa["skip"]["W"]], axis=0).T
        bkqvs = jnp.concatenate(
            [jnp.zeros((_D,), jnp.float32), jnp.zeros((_D,), jnp.float32),
             pa["value"]["b"], pa["skip"]["b"]])
        kskip, qv = _atom_proj(x, wkqvs, bkqvs, br=2000)
        kk, skip_o = kskip[:, :_D], kskip[:, _D:]
        e_bias = pa["edge"]["b"] + pa["key"]["b"] + pa["query"]["b"]
        e = _linear(edge_attr, pa["edge"]["W"].T, e_bias, br=2000)
        msg = _sc_atom_msg(kk, qv, e, dst, src)
        agg = _sc_scatter_add(msg, dst, zrows)
        x = _atom_post(x, skip_o, agg, pa["bn_gamma"], pa["bn_beta"])

        w12 = jnp.concatenate([pb["v1"]["W"], pb["v2"]["W"]], axis=0).T
        a12 = _linear(x, w12, None, br=2000)
        a1, a2 = a12[:, :_D], a12[:, _D:]
        b0 = pb["v0"]["b"] + pb["v1"]["b"] + pb["v2"]["b"]
        a1g, a2g = _sc_gather((a1, a2), (src, dst), (0, 1))
        s, stats = _bond_sum_stats(edge_attr, pb["v0"]["W"].T, b0, a1g, a2g)
        edge_attr = _bond_post(edge_attr, s, stats, pb["bn_gamma"],
                               pb["bn_beta"])

    boa = _atom_head(x, params["atom_mlp"]["l1"]["W"].T,
                     params["atom_mlp"]["l1"]["b"],
                     params["atom_mlp"]["l2"]["W"].T,
                     params["atom_mlp"]["l2"]["b"])
    bonds = _bonds_head(edge_attr, params["bond_mlp"]["l1"]["W"].T,
                        params["bond_mlp"]["l1"]["b"],
                        params["bond_mlp"]["l2"]["W"].T,
                        params["bond_mlp"]["l2"]["b"])
    return boa.reshape(-1, 8, 100), bonds


# speedup vs baseline: 3.6897x; 1.1012x over previous
import functools

import jax
import jax.numpy as jnp
from jax import lax
from jax.experimental import pallas as pl
from jax.experimental.pallas import tpu as pltpu
from jax.experimental.pallas import tpu_sc as plsc

_N = 10000
_E = 160000
_D = 128
_NC = 2
_NS = 16
_NW = _NC * _NS
_RPW = _E // _NW
_CH = 128
_NFULL = _RPW // _CH
_TAIL = _RPW - _NFULL * _CH
_NPAD = 10240
_RPT = _NPAD // _NS


def _sc_mesh():
    return plsc.VectorSubcoreMesh(core_axis_name="c", subcore_axis_name="s",
                                  num_cores=_NC, num_subcores=_NS)


_NBUF = 4
_SNBUF = 2
_CHUNKS = [(c * _CH, _CH) for c in range(_NFULL)] + [(_NFULL * _CH, _TAIL)]


def _sc_gather(tables, idxs, idx_of):
    n_t = len(tables)
    n_i = len(idxs)
    width = tables[0].shape[1]
    dt = tables[0].dtype
    scratch = (
        [pltpu.VMEM((_RPW,), jnp.int32) for _ in range(n_i)]
        + [pltpu.VMEM((_CH, width), dt) for _ in range(_NBUF)]
        + [pltpu.SemaphoreType.DMA, pltpu.SemaphoreType.DMA]
    )
    out_type = tuple(
        jax.ShapeDtypeStruct((_E, width), dt) for _ in range(n_t))

    def body(*refs):
        tab = refs[:n_t]
        idx = refs[n_t:n_t + n_i]
        outs = refs[n_t + n_i:n_t + n_i + n_t]
        s = n_t + n_i + n_t
        ib = refs[s:s + n_i]; s += n_i
        rb = refs[s:s + _NBUF]; s += _NBUF
        sem_g, sem_w = refs[s], refs[s + 1]
        wid = lax.axis_index("s") * _NC + lax.axis_index("c")
        base = wid * _RPW
        for j in range(n_i):
            pltpu.sync_copy(idx[j].at[pl.ds(base, _RPW)], ib[j])

        jobs = [(t, off, sz) for t in range(n_t) for (off, sz) in _CHUNKS]
        pend_w = [None] * _NBUF
        prev = None

        def buf(b, sz):
            return rb[b] if sz == _CH else rb[b].at[pl.ds(0, sz)]

        def flush_prev():
            pb, pt, poff, psz, pg = prev
            pg.wait()
            pend_w[pb] = pltpu.async_copy(
                buf(pb, psz), outs[pt].at[pl.ds(base + poff, psz)], sem_w)

        for c, (t, off, sz) in enumerate(jobs):
            b = c % _NBUF
            if pend_w[b] is not None:
                pend_w[b].wait()
                pend_w[b] = None
            g = pltpu.async_copy(
                tab[t].at[ib[idx_of[t]].at[pl.ds(off, sz)]], buf(b, sz), sem_g)
            if prev is not None:
                flush_prev()
            prev = (b, t, off, sz, g)
        flush_prev()
        for d in pend_w:
            if d is not None:
                d.wait()

    f = pl.kernel(body, out_type=out_type, mesh=_sc_mesh(),
                  scratch_types=scratch)
    return f(*tables, *idxs)


def _sc_atom_msg(kk, qv, e, dst, src):
    scratch = (
        [pltpu.VMEM((_RPW,), jnp.int32) for _ in range(2)]
        + [pltpu.VMEM((_CH, _D), jnp.float32) for _ in range(2)]
        + [pltpu.VMEM((_CH, _D), jnp.int32) for _ in range(2)]
        + [pltpu.VMEM((_CH, _D), jnp.float32) for _ in range(2)]
        + [pltpu.SemaphoreType.DMA for _ in range(4)]
    )

    def body(kk_h, qv_h, e_h, dst_h, src_h, out_h, *rest):
        ibd, ibs = rest[0], rest[1]
        bk = rest[2:4]
        bqv = rest[4:6]
        be = rest[6:8]
        sem_g = rest[8:10]
        sem_w = rest[10:12]
        wid = lax.axis_index("s") * _NC + lax.axis_index("c")
        base = wid * _RPW
        pltpu.sync_copy(dst_h.at[pl.ds(base, _RPW)], ibd)
        pltpu.sync_copy(src_h.at[pl.ds(base, _RPW)], ibs)

        def sl(ref, sz):
            return ref if sz == _CH else ref.at[pl.ds(0, sz)]

        pend_w = [None, None]
        prev = None

        def compute_and_flush():
            b, off, sz, descs = prev
            for d_ in descs:
                d_.wait()

            def row(r, carry):
                for j in range(_D // 16):
                    s_ = pl.ds(j * 16, 16)
                    x_ = bqv[b][r, s_]
                    qf = ((x_ << 16) >> 16).astype(jnp.float32) * (1.0 / 256.0)
                    vf = (x_ >> 16).astype(jnp.float32) * (1.0 / 256.0)
                    xv = bk[b][r, s_] + qf + be[b][r, s_]
                    be[b][r, s_] = vf / (1.0 + jnp.exp(-xv))
                return carry

            lax.fori_loop(0, sz, row, 0)
            pend_w[b] = pltpu.async_copy(
                sl(be[b], sz), out_h.at[pl.ds(base + off, sz)], sem_w[b])

        for c, (off, sz) in enumerate(_CHUNKS):
            b = c % 2
            if pend_w[b] is not None:
                pend_w[b].wait()
                pend_w[b] = None
            ds_ = ibd.at[pl.ds(off, sz)]
            ss_ = ibs.at[pl.ds(off, sz)]
            descs = [
                pltpu.async_copy(kk_h.at[ds_], sl(bk[b], sz), sem_g[b]),
                pltpu.async_copy(qv_h.at[ss_], sl(bqv[b], sz), sem_g[b]),
                pltpu.async_copy(e_h.at[pl.ds(base + off, sz)],
                                 sl(be[b], sz), sem_g[b]),
            ]
            if prev is not None:
                compute_and_flush()
            prev = (b, off, sz, descs)
        compute_and_flush()
        for d_ in pend_w:
            if d_ is not None:
                d_.wait()

    f = pl.kernel(body, out_type=jax.ShapeDtypeStruct((_E, _D), jnp.float32),
                  mesh=_sc_mesh(), scratch_types=scratch)
    return f(kk, qv, e, dst, src)


def _sc_scatter_add(msg, dst, zrows):
    scratch = (
        [pltpu.VMEM((_CH,), jnp.int32) for _ in range(_SNBUF)]
        + [pltpu.VMEM((_CH, _D), jnp.float32) for _ in range(_SNBUF)]
        + [pltpu.VMEM((_TAIL,), jnp.int32),
           pltpu.VMEM((_TAIL, _D), jnp.float32),
           pltpu.VMEM_SHARED((_NPAD, _D), jnp.float32),
           pltpu.SemaphoreType.DMA, pltpu.SemaphoreType.DMA]
    )

    def body(msg_hbm, dst_hbm, z_hbm, out_hbm, *rest):
        ib = rest[:_SNBUF]
        rb = rest[_SNBUF:2 * _SNBUF]
        ib_t, rb_t, acc, sem_l, sem_s = rest[2 * _SNBUF:]
        cid = lax.axis_index("c")
        sid = lax.axis_index("s")
        wid = sid * _NC + cid
        base = wid * _RPW
        pltpu.sync_copy(z_hbm, acc.at[pl.ds(sid * _RPT, _RPT)])
        plsc.subcore_barrier()

        pend_s = [None] * (_SNBUF + 1)
        prev = None

        def flush_prev():
            slot, pib, prb, li, lm = prev
            li.wait()
            lm.wait()
            pend_s[slot] = pltpu.async_copy(prb, acc.at[pib], sem_s, add=True)

        for c, (off, sz) in enumerate(_CHUNKS):
            if sz == _CH:
                slot = c % _SNBUF
                cib, crb = ib[slot], rb[slot]
            else:
                slot = _SNBUF
                cib, crb = ib_t, rb_t
            if pend_s[slot] is not None:
                pend_s[slot].wait()
                pend_s[slot] = None
            li = pltpu.async_copy(dst_hbm.at[pl.ds(base + off, sz)], cib, sem_l)
            lm = pltpu.async_copy(msg_hbm.at[pl.ds(base + off, sz)], crb, sem_l)
            if prev is not None:
                flush_prev()
            prev = (slot, cib, crb, li, lm)
        flush_prev()
        for d in pend_s:
            if d is not None:
                d.wait()
        plsc.subcore_barrier()
        pltpu.sync_copy(acc.at[pl.ds(sid * _RPT, _RPT)],
                        out_hbm.at[cid, pl.ds(sid * _RPT, _RPT)])

    f = pl.kernel(body,
                  out_type=jax.ShapeDtypeStruct((_NC, _NPAD, _D), jnp.float32),
                  mesh=_sc_mesh(), scratch_types=scratch)
    return f(msg, dst, zrows)


def _linear(x, wt, b=None, act=None, br=4000, out_dtype=jnp.float32):
    r, d = x.shape
    f_out = wt.shape[1]
    assert r % br == 0
    grid = (r // br,)
    in_specs = [pl.BlockSpec((br, d), lambda i: (i, 0)),
                pl.BlockSpec((d, f_out), lambda i: (0, 0))]
    args = [x, wt]
    if b is not None:
        in_specs.append(pl.BlockSpec((1, f_out), lambda i: (0, 0)))
        args.append(b.reshape(1, f_out))

    def body(x_ref, w_ref, *rest):
        if b is not None:
            b_ref, o_ref = rest
        else:
            (o_ref,) = rest
        y = jnp.dot(x_ref[...], w_ref[...], preferred_element_type=jnp.float32)
        if b is not None:
            y = y + b_ref[...]
        if act == "relu":
            y = jnp.maximum(y, 0.0)
        o_ref[...] = y.astype(out_dtype)

    return pl.pallas_call(
        body, grid=grid, in_specs=in_specs,
        out_specs=pl.BlockSpec((br, f_out), lambda i: (i, 0)),
        out_shape=jax.ShapeDtypeStruct((r, f_out), out_dtype))(*args)


def _atom_post(x, skip_o, agg, gamma, beta):
    def body(x_r, s_r, a_r, g_r, b_r, o_r):
        h = s_r[...] + a_r[0, :_N] + a_r[1, :_N]
        mu = jnp.mean(h, axis=0, keepdims=True)
        var = jnp.mean((h - mu) ** 2, axis=0, keepdims=True)
        hn = g_r[...] * (h - mu) * lax.rsqrt(var + 1e-5) + b_r[...]
        o_r[...] = x_r[...] + jnp.maximum(hn, 0.0)

    return pl.pallas_call(
        body, out_shape=jax.ShapeDtypeStruct((_N, _D), jnp.float32))(
            x, skip_o, agg, gamma.reshape(1, _D), beta.reshape(1, _D))


def _bond_sum_stats(ea, wt, b, a1g, a2g, br=4000):
    grid = (_E // br,)
    spec = pl.BlockSpec((br, _D), lambda i: (i, 0))
    wspec = pl.BlockSpec((_D, _D), lambda i: (0, 0))
    bspec = pl.BlockSpec((1, _D), lambda i: (0, 0))

    def body(ea_r, w_r, b_r, a1_r, a2_r, s_r, st_r, sacc, qacc):
        i = pl.program_id(0)
        v = (jnp.dot(ea_r[...], w_r[...], preferred_element_type=jnp.float32)
             + b_r[...] + a1_r[...] + a2_r[...])
        s_r[...] = v

        @pl.when(i == 0)
        def _():
            sacc[...] = jnp.zeros_like(sacc)
            qacc[...] = jnp.zeros_like(qacc)

        sacc[...] += jnp.sum(v, axis=0).reshape(1, _D)
        qacc[...] += jnp.sum(v * v, axis=0).reshape(1, _D)

        @pl.when(i == grid[0] - 1)
        def _():
            st_r[0:1, :] = sacc[...]
            st_r[1:2, :] = qacc[...]

    return pl.pallas_call(
        body, grid=grid, in_specs=[spec, wspec, bspec, spec, spec],
        out_specs=(spec, pl.BlockSpec((2, _D), lambda i: (0, 0))),
        out_shape=(jax.ShapeDtypeStruct((_E, _D), jnp.float32),
                   jax.ShapeDtypeStruct((2, _D), jnp.float32)),
        scratch_shapes=[pltpu.VMEM((1, _D), jnp.float32),
                        pltpu.VMEM((1, _D), jnp.float32)])(
            ea, wt, b.reshape(1, _D), a1g, a2g)


def _bond_post(ea, s, stats, gamma, beta, br=4000):
    grid = (_E // br,)
    spec = pl.BlockSpec((br, _D), lambda i: (i, 0))
    one = pl.BlockSpec((1, _D), lambda i: (0, 0))

    def body(ea_r, s_r, st_r, g_r, b_r, o_r):
        mu = st_r[0:1, :] * (1.0 / _E)
        var = st_r[1:2, :] * (1.0 / _E) - mu * mu
        hn = g_r[...] * (s_r[...] - mu) * lax.rsqrt(var + 1e-5) + b_r[...]
        o_r[...] = ea_r[...] + jnp.maximum(hn, 0.0)

    return pl.pallas_call(
        body, grid=grid,
        in_specs=[spec, spec, pl.BlockSpec((2, _D), lambda i: (0, 0)), one, one],
        out_specs=spec,
        out_shape=jax.ShapeDtypeStruct((_E, _D), jnp.float32))(
            ea, s, stats, gamma.reshape(1, _D), beta.reshape(1, _D))


def _atom_proj(x, wt, b, br=4000):
    grid = (_N // br,)

    def body(x_ref, w_ref, b_ref, ks_ref, qv_ref):
        y = jnp.dot(x_ref[...], w_ref[...],
                    preferred_element_type=jnp.float32) + b_ref[...]
        ks_ref[:, :_D] = y[:, :_D]
        ks_ref[:, _D:] = y[:, 3 * _D:]
        qi = jnp.clip(jnp.round(y[:, _D:2 * _D] * 256.0), -32768.0,
                      32767.0).astype(jnp.int32)
        vi = jnp.clip(jnp.round(y[:, 2 * _D:3 * _D] * 256.0), -32768.0,
                      32767.0).astype(jnp.int32)
        qv_ref[...] = (vi << 16) | (qi & 0xFFFF)

    return pl.pallas_call(
        body, grid=grid,
        in_specs=[pl.BlockSpec((br, _D), lambda i: (i, 0)),
                  pl.BlockSpec((_D, 4 * _D), lambda i: (0, 0)),
                  pl.BlockSpec((1, 4 * _D), lambda i: (0, 0))],
        out_specs=(pl.BlockSpec((br, 2 * _D), lambda i: (i, 0)),
                   pl.BlockSpec((br, _D), lambda i: (i, 0))),
        out_shape=(jax.ShapeDtypeStruct((_N, 2 * _D), jnp.float32),
                   jax.ShapeDtypeStruct((_N, _D), jnp.int32)))(
            x, wt, b.reshape(1, 4 * _D))


def _bonds_head(ea, w1t, b1, w2t, b2, br=4000):
    grid = (_E // br,)
    f_out = w2t.shape[1]
    spec = pl.BlockSpec((br, _D), lambda i: (i, 0))

    def body(ea_r, w1_r, b1_r, w2_r, b2_r, o_r):
        h = jnp.maximum(
            jnp.dot(ea_r[...], w1_r[...], preferred_element_type=jnp.float32)
            + b1_r[...], 0.0)
        o_r[...] = jnp.dot(h, w2_r[...],
                           preferred_element_type=jnp.float32) + b2_r[...]

    return pl.pallas_call(
        body, grid=grid,
        in_specs=[spec, pl.BlockSpec((_D, _D), lambda i: (0, 0)),
                  pl.BlockSpec((1, _D), lambda i: (0, 0)),
                  pl.BlockSpec((_D, f_out), lambda i: (0, 0)),
                  pl.BlockSpec((1, f_out), lambda i: (0, 0))],
        out_specs=pl.BlockSpec((br, f_out), lambda i: (i, 0)),
        out_shape=jax.ShapeDtypeStruct((_E, f_out), jnp.float32))(
            ea, w1t, b1.reshape(1, _D), w2t, b2.reshape(1, f_out))


def _atom_head(x, w1t, b1, w2t, b2):
    f_out = w2t.shape[1]

    def body(x_r, w1_r, b1_r, w2_r, b2_r, o_r):
        z = jnp.mean(x_r[...], axis=0, keepdims=True)
        h = jnp.maximum(
            jnp.dot(z, w1_r[...], preferred_element_type=jnp.float32)
            + b1_r[...], 0.0)
        o_r[...] = jnp.dot(h, w2_r[...],
                           preferred_element_type=jnp.float32) + b2_r[...]

    return pl.pallas_call(
        body, out_shape=jax.ShapeDtypeStruct((1, f_out), jnp.float32))(
            x, w1t, b1.reshape(1, -1), w2t, b2.reshape(1, -1))


def kernel(x, edge_index, edge_attr, params):
    src = edge_index[0]
    dst = edge_index[1]
    zrows = jnp.zeros((_RPT, _D), jnp.float32)

    for pa, pb in zip(params["atom_layers"], params["bond_layers"]):
        wkqvs = jnp.concatenate(
            [pa["key"]["W"], pa["query"]["W"], pa["value"]["W"],
             pa["skip"]["W"]], axis=0).T
        bkqvs = jnp.concatenate(
            [jnp.zeros((_D,), jnp.float32), jnp.zeros((_D,), jnp.float32),
             pa["value"]["b"], pa["skip"]["b"]])
        kskip, qv = _atom_proj(x, wkqvs, bkqvs, br=2000)
        kk, skip_o = kskip[:, :_D], kskip[:, _D:]
        e_bias = pa["edge"]["b"] + pa["key"]["b"] + pa["query"]["b"]
        e = _linear(edge_attr, pa["edge"]["W"].T, e_bias, br=4000)
        msg = _sc_atom_msg(kk, qv, e, dst, src)
        agg = _sc_scatter_add(msg, dst, zrows)
        x = _atom_post(x, skip_o, agg, pa["bn_gamma"], pa["bn_beta"])

        w12 = jnp.concatenate([pb["v1"]["W"], pb["v2"]["W"]], axis=0).T
        a12 = _linear(x, w12, None, br=2000)
        a1, a2 = a12[:, :_D], a12[:, _D:]
        b0 = pb["v0"]["b"] + pb["v1"]["b"] + pb["v2"]["b"]
        a1g, a2g = _sc_gather((a1, a2), (src, dst), (0, 1))
        s, stats = _bond_sum_stats(edge_attr, pb["v0"]["W"].T, b0, a1g, a2g)
        edge_attr = _bond_post(edge_attr, s, stats, pb["bn_gamma"],
                               pb["bn_beta"])

    boa = _atom_head(x, params["atom_mlp"]["l1"]["W"].T,
                     params["atom_mlp"]["l1"]["b"],
                     params["atom_mlp"]["l2"]["W"].T,
                     params["atom_mlp"]["l2"]["b"])
    bonds = _bonds_head(edge_attr, params["bond_mlp"]["l1"]["W"].T,
                        params["bond_mlp"]["l1"]["b"],
                        params["bond_mlp"]["l2"]["W"].T,
                        params["bond_mlp"]["l2"]["b"])
    return boa.reshape(-1, 8, 100), bonds


# speedup vs baseline: 3.8664x; 1.0479x over previous
import functools

import jax
import jax.numpy as jnp
from jax import lax
from jax.experimental import pallas as pl
from jax.experimental.pallas import tpu as pltpu
from jax.experimental.pallas import tpu_sc as plsc

_N = 10000
_E = 160000
_D = 128
_NC = 2
_NS = 16
_NW = _NC * _NS
_RPW = _E // _NW
_CH = 128
_NFULL = _RPW // _CH
_TAIL = _RPW - _NFULL * _CH
_NPAD = 10240
_RPT = _NPAD // _NS


def _sc_mesh():
    return plsc.VectorSubcoreMesh(core_axis_name="c", subcore_axis_name="s",
                                  num_cores=_NC, num_subcores=_NS)


_NBUF = 4
_SNBUF = 3
_SCH = 96
_SCHUNKS = ([(c * _SCH, _SCH) for c in range(_RPW // _SCH)]
            + [((_RPW // _SCH) * _SCH, _RPW - (_RPW // _SCH) * _SCH)])
_CHUNKS = [(c * _CH, _CH) for c in range(_NFULL)] + [(_NFULL * _CH, _TAIL)]


def _sc_gather(tables, idxs, idx_of):
    n_t = len(tables)
    n_i = len(idxs)
    width = tables[0].shape[1]
    dt = tables[0].dtype
    scratch = (
        [pltpu.VMEM((_RPW,), jnp.int32) for _ in range(n_i)]
        + [pltpu.VMEM((_CH, width), dt) for _ in range(_NBUF)]
        + [pltpu.SemaphoreType.DMA, pltpu.SemaphoreType.DMA]
    )
    out_type = tuple(
        jax.ShapeDtypeStruct((_E, width), dt) for _ in range(n_t))

    def body(*refs):
        tab = refs[:n_t]
        idx = refs[n_t:n_t + n_i]
        outs = refs[n_t + n_i:n_t + n_i + n_t]
        s = n_t + n_i + n_t
        ib = refs[s:s + n_i]; s += n_i
        rb = refs[s:s + _NBUF]; s += _NBUF
        sem_g, sem_w = refs[s], refs[s + 1]
        wid = lax.axis_index("s") * _NC + lax.axis_index("c")
        base = wid * _RPW
        for j in range(n_i):
            pltpu.sync_copy(idx[j].at[pl.ds(base, _RPW)], ib[j])

        jobs = [(t, off, sz) for t in range(n_t) for (off, sz) in _CHUNKS]
        pend_w = [None] * _NBUF
        prev = None

        def buf(b, sz):
            return rb[b] if sz == _CH else rb[b].at[pl.ds(0, sz)]

        def flush_prev():
            pb, pt, poff, psz, pg = prev
            pg.wait()
            pend_w[pb] = pltpu.async_copy(
                buf(pb, psz), outs[pt].at[pl.ds(base + poff, psz)], sem_w)

        for c, (t, off, sz) in enumerate(jobs):
            b = c % _NBUF
            if pend_w[b] is not None:
                pend_w[b].wait()
                pend_w[b] = None
            g = pltpu.async_copy(
                tab[t].at[ib[idx_of[t]].at[pl.ds(off, sz)]], buf(b, sz), sem_g)
            if prev is not None:
                flush_prev()
            prev = (b, t, off, sz, g)
        flush_prev()
        for d in pend_w:
            if d is not None:
                d.wait()

    f = pl.kernel(body, out_type=out_type, mesh=_sc_mesh(),
                  scratch_types=scratch)
    return f(*tables, *idxs)


def _sc_atom_msg(kk, qv, e, dst, src):
    scratch = (
        [pltpu.VMEM((_RPW,), jnp.int32) for _ in range(2)]
        + [pltpu.VMEM((_CH, _D), jnp.float32) for _ in range(2)]
        + [pltpu.VMEM((_CH, _D), jnp.int32) for _ in range(2)]
        + [pltpu.VMEM((_CH, _D), jnp.float32) for _ in range(2)]
        + [pltpu.SemaphoreType.DMA for _ in range(4)]
    )

    def body(kk_h, qv_h, e_h, dst_h, src_h, out_h, *rest):
        ibd, ibs = rest[0], rest[1]
        bk = rest[2:4]
        bqv = rest[4:6]
        be = rest[6:8]
        sem_g = rest[8:10]
        sem_w = rest[10:12]
        wid = lax.axis_index("s") * _NC + lax.axis_index("c")
        base = wid * _RPW
        pltpu.sync_copy(dst_h.at[pl.ds(base, _RPW)], ibd)
        pltpu.sync_copy(src_h.at[pl.ds(base, _RPW)], ibs)

        def sl(ref, sz):
            return ref if sz == _CH else ref.at[pl.ds(0, sz)]

        pend_w = [None, None]
        prev = None

        def compute_and_flush():
            b, off, sz, descs = prev
            for d_ in descs:
                d_.wait()

            def row(r, carry):
                for j in range(_D // 16):
                    s_ = pl.ds(j * 16, 16)
                    x_ = bqv[b][r, s_]
                    qf = ((x_ << 16) >> 16).astype(jnp.float32) * (1.0 / 256.0)
                    vf = (x_ >> 16).astype(jnp.float32) * (1.0 / 256.0)
                    xv = bk[b][r, s_] + qf + be[b][r, s_]
                    be[b][r, s_] = vf / (1.0 + jnp.exp(-xv))
                return carry

            lax.fori_loop(0, sz, row, 0)
            pend_w[b] = pltpu.async_copy(
                sl(be[b], sz), out_h.at[pl.ds(base + off, sz)], sem_w[b])

        for c, (off, sz) in enumerate(_CHUNKS):
            b = c % 2
            if pend_w[b] is not None:
                pend_w[b].wait()
                pend_w[b] = None
            ds_ = ibd.at[pl.ds(off, sz)]
            ss_ = ibs.at[pl.ds(off, sz)]
            descs = [
                pltpu.async_copy(kk_h.at[ds_], sl(bk[b], sz), sem_g[b]),
                pltpu.async_copy(qv_h.at[ss_], sl(bqv[b], sz), sem_g[b]),
                pltpu.async_copy(e_h.at[pl.ds(base + off, sz)],
                                 sl(be[b], sz), sem_g[b]),
            ]
            if prev is not None:
                compute_and_flush()
            prev = (b, off, sz, descs)
        compute_and_flush()
        for d_ in pend_w:
            if d_ is not None:
                d_.wait()

    f = pl.kernel(body, out_type=jax.ShapeDtypeStruct((_E, _D), jnp.float32),
                  mesh=_sc_mesh(), scratch_types=scratch)
    return f(kk, qv, e, dst, src)


def _sc_scatter_add(msg, dst, zrows):
    scratch = (
        [pltpu.VMEM((_SCH,), jnp.int32) for _ in range(_SNBUF)]
        + [pltpu.VMEM((_SCH, _D), jnp.float32) for _ in range(_SNBUF)]
        + [pltpu.VMEM((_RPW - (_RPW // _SCH) * _SCH,), jnp.int32),
           pltpu.VMEM((_RPW - (_RPW // _SCH) * _SCH, _D), jnp.float32),
           pltpu.VMEM_SHARED((_NPAD, _D), jnp.float32),
           pltpu.SemaphoreType.DMA, pltpu.SemaphoreType.DMA]
    )

    def body(msg_hbm, dst_hbm, z_hbm, out_hbm, *rest):
        ib = rest[:_SNBUF]
        rb = rest[_SNBUF:2 * _SNBUF]
        ib_t, rb_t, acc, sem_l, sem_s = rest[2 * _SNBUF:]
        cid = lax.axis_index("c")
        sid = lax.axis_index("s")
        wid = sid * _NC + cid
        base = wid * _RPW
        pltpu.sync_copy(z_hbm, acc.at[pl.ds(sid * _RPT, _RPT)])
        plsc.subcore_barrier()

        pend_s = [None] * (_SNBUF + 1)
        prev = None

        def flush_prev():
            slot, pib, prb, li, lm = prev
            li.wait()
            lm.wait()
            pend_s[slot] = pltpu.async_copy(prb, acc.at[pib], sem_s, add=True)

        for c, (off, sz) in enumerate(_SCHUNKS):
            if sz == _SCH:
                slot = c % _SNBUF
                cib, crb = ib[slot], rb[slot]
            else:
                slot = _SNBUF
                cib, crb = ib_t, rb_t
            if pend_s[slot] is not None:
                pend_s[slot].wait()
                pend_s[slot] = None
            li = pltpu.async_copy(dst_hbm.at[pl.ds(base + off, sz)], cib, sem_l)
            lm = pltpu.async_copy(msg_hbm.at[pl.ds(base + off, sz)], crb, sem_l)
            if prev is not None:
                flush_prev()
            prev = (slot, cib, crb, li, lm)
        flush_prev()
        for d in pend_s:
            if d is not None:
                d.wait()
        plsc.subcore_barrier()
        pltpu.sync_copy(acc.at[pl.ds(sid * _RPT, _RPT)],
                        out_hbm.at[cid, pl.ds(sid * _RPT, _RPT)])

    f = pl.kernel(body,
                  out_type=jax.ShapeDtypeStruct((_NC, _NPAD, _D), jnp.float32),
                  mesh=_sc_mesh(), scratch_types=scratch)
    return f(msg, dst, zrows)


def _linear(x, wt, b=None, act=None, br=4000, out_dtype=jnp.float32):
    r, d = x.shape
    f_out = wt.shape[1]
    assert r % br == 0
    grid = (r // br,)
    in_specs = [pl.BlockSpec((br, d), lambda i: (i, 0)),
                pl.BlockSpec((d, f_out), lambda i: (0, 0))]
    args = [x, wt]
    if b is not None:
        in_specs.append(pl.BlockSpec((1, f_out), lambda i: (0, 0)))
        args.append(b.reshape(1, f_out))

    def body(x_ref, w_ref, *rest):
        if b is not None:
            b_ref, o_ref = rest
        else:
            (o_ref,) = rest
        y = jnp.dot(x_ref[...], w_ref[...], preferred_element_type=jnp.float32)
        if b is not None:
            y = y + b_ref[...]
        if act == "relu":
            y = jnp.maximum(y, 0.0)
        o_ref[...] = y.astype(out_dtype)

    return pl.pallas_call(
        body, grid=grid, in_specs=in_specs,
        out_specs=pl.BlockSpec((br, f_out), lambda i: (i, 0)),
        out_shape=jax.ShapeDtypeStruct((r, f_out), out_dtype))(*args)


def _atom_post(x, skip_o, agg, gamma, beta):
    def body(x_r, s_r, a_r, g_r, b_r, o_r):
        h = s_r[...] + a_r[0, :_N] + a_r[1, :_N]
        mu = jnp.mean(h, axis=0, keepdims=True)
        var = jnp.mean((h - mu) ** 2, axis=0, keepdims=True)
        hn = g_r[...] * (h - mu) * lax.rsqrt(var + 1e-5) + b_r[...]
        o_r[...] = x_r[...] + jnp.maximum(hn, 0.0)

    return pl.pallas_call(
        body, out_shape=jax.ShapeDtypeStruct((_N, _D), jnp.float32))(
            x, skip_o, agg, gamma.reshape(1, _D), beta.reshape(1, _D))


def _bond_sum_stats(ea, wt, b, a1g, a2g, br=4000):
    grid = (_E // br,)
    spec = pl.BlockSpec((br, _D), lambda i: (i, 0))
    wspec = pl.BlockSpec((_D, _D), lambda i: (0, 0))
    bspec = pl.BlockSpec((1, _D), lambda i: (0, 0))

    def body(ea_r, w_r, b_r, a1_r, a2_r, s_r, st_r, sacc, qacc):
        i = pl.program_id(0)
        v = (jnp.dot(ea_r[...], w_r[...], preferred_element_type=jnp.float32)
             + b_r[...] + a1_r[...] + a2_r[...])
        s_r[...] = v

        @pl.when(i == 0)
        def _():
            sacc[...] = jnp.zeros_like(sacc)
            qacc[...] = jnp.zeros_like(qacc)

        sacc[...] += jnp.sum(v, axis=0).reshape(1, _D)
        qacc[...] += jnp.sum(v * v, axis=0).reshape(1, _D)

        @pl.when(i == grid[0] - 1)
        def _():
            st_r[0:1, :] = sacc[...]
            st_r[1:2, :] = qacc[...]

    return pl.pallas_call(
        body, grid=grid, in_specs=[spec, wspec, bspec, spec, spec],
        out_specs=(spec, pl.BlockSpec((2, _D), lambda i: (0, 0))),
        out_shape=(jax.ShapeDtypeStruct((_E, _D), jnp.float32),
                   jax.ShapeDtypeStruct((2, _D), jnp.float32)),
        scratch_shapes=[pltpu.VMEM((1, _D), jnp.float32),
                        pltpu.VMEM((1, _D), jnp.float32)])(
            ea, wt, b.reshape(1, _D), a1g, a2g)


def _bond_post(ea, s, stats, gamma, beta, we_next=None, be_next=None,
               br=4000):
    grid = (_E // br,)
    spec = pl.BlockSpec((br, _D), lambda i: (i, 0))
    one = pl.BlockSpec((1, _D), lambda i: (0, 0))
    fused = we_next is not None

    def body(ea_r, s_r, st_r, g_r, b_r, *rest):
        mu = st_r[0:1, :] * (1.0 / _E)
        var = st_r[1:2, :] * (1.0 / _E) - mu * mu
        hn = g_r[...] * (s_r[...] - mu) * lax.rsqrt(var + 1e-5) + b_r[...]
        ea_new = ea_r[...] + jnp.maximum(hn, 0.0)
        if fused:
            w_r, eb_r, o_r, e_r = rest
            e_r[...] = jnp.dot(ea_new, w_r[...],
                               preferred_element_type=jnp.float32) + eb_r[...]
        else:
            (o_r,) = rest
        o_r[...] = ea_new

    in_specs = [spec, spec, pl.BlockSpec((2, _D), lambda i: (0, 0)), one, one]
    args = [ea, s, stats, gamma.reshape(1, _D), beta.reshape(1, _D)]
    if fused:
        in_specs += [pl.BlockSpec((_D, _D), lambda i: (0, 0)), one]
        args += [we_next, be_next.reshape(1, _D)]
        return pl.pallas_call(
            body, grid=grid, in_specs=in_specs, out_specs=(spec, spec),
            out_shape=(jax.ShapeDtypeStruct((_E, _D), jnp.float32),
                       jax.ShapeDtypeStruct((_E, _D), jnp.float32)))(*args)
    return pl.pallas_call(
        body, grid=grid, in_specs=in_specs, out_specs=spec,
        out_shape=jax.ShapeDtypeStruct((_E, _D), jnp.float32))(*args)


def _atom_proj(x, wt, b, br=4000):
    grid = (_N // br,)

    def body(x_ref, w_ref, b_ref, ks_ref, qv_ref):
        y = jnp.dot(x_ref[...], w_ref[...],
                    preferred_element_type=jnp.float32) + b_ref[...]
        ks_ref[:, :_D] = y[:, :_D]
        ks_ref[:, _D:] = y[:, 3 * _D:]
        qi = jnp.clip(jnp.round(y[:, _D:2 * _D] * 256.0), -32768.0,
                      32767.0).astype(jnp.int32)
        vi = jnp.clip(jnp.round(y[:, 2 * _D:3 * _D] * 256.0), -32768.0,
                      32767.0).astype(jnp.int32)
        qv_ref[...] = (vi << 16) | (qi & 0xFFFF)

    return pl.pallas_call(
        body, grid=grid,
        in_specs=[pl.BlockSpec((br, _D), lambda i: (i, 0)),
                  pl.BlockSpec((_D, 4 * _D), lambda i: (0, 0)),
                  pl.BlockSpec((1, 4 * _D), lambda i: (0, 0))],
        out_specs=(pl.BlockSpec((br, 2 * _D), lambda i: (i, 0)),
                   pl.BlockSpec((br, _D), lambda i: (i, 0))),
        out_shape=(jax.ShapeDtypeStruct((_N, 2 * _D), jnp.float32),
                   jax.ShapeDtypeStruct((_N, _D), jnp.int32)))(
            x, wt, b.reshape(1, 4 * _D))


def _bonds_head(ea, w1t, b1, w2t, b2, br=4000):
    grid = (_E // br,)
    f_out = w2t.shape[1]
    spec = pl.BlockSpec((br, _D), lambda i: (i, 0))

    def body(ea_r, w1_r, b1_r, w2_r, b2_r, o_r):
        h = jnp.maximum(
            jnp.dot(ea_r[...], w1_r[...], preferred_element_type=jnp.float32)
            + b1_r[...], 0.0)
        o_r[...] = jnp.dot(h, w2_r[...],
                           preferred_element_type=jnp.float32) + b2_r[...]

    return pl.pallas_call(
        body, grid=grid,
        in_specs=[spec, pl.BlockSpec((_D, _D), lambda i: (0, 0)),
                  pl.BlockSpec((1, _D), lambda i: (0, 0)),
                  pl.BlockSpec((_D, f_out), lambda i: (0, 0)),
                  pl.BlockSpec((1, f_out), lambda i: (0, 0))],
        out_specs=pl.BlockSpec((br, f_out), lambda i: (i, 0)),
        out_shape=jax.ShapeDtypeStruct((_E, f_out), jnp.float32))(
            ea, w1t, b1.reshape(1, _D), w2t, b2.reshape(1, f_out))


def _atom_head(x, w1t, b1, w2t, b2):
    f_out = w2t.shape[1]

    def body(x_r, w1_r, b1_r, w2_r, b2_r, o_r):
        z = jnp.mean(x_r[...], axis=0, keepdims=True)
        h = jnp.maximum(
            jnp.dot(z, w1_r[...], preferred_element_type=jnp.float32)
            + b1_r[...], 0.0)
        o_r[...] = jnp.dot(h, w2_r[...],
                           preferred_element_type=jnp.float32) + b2_r[...]

    return pl.pallas_call(
        body, out_shape=jax.ShapeDtypeStruct((1, f_out), jnp.float32))(
            x, w1t, b1.reshape(1, -1), w2t, b2.reshape(1, -1))


def kernel(x, edge_index, edge_attr, params):
    src = edge_index[0]
    dst = edge_index[1]
    zrows = jnp.zeros((_RPT, _D), jnp.float32)

    atom_layers = params["atom_layers"]
    n_layers = len(atom_layers)

    def ew_bias(pa):
        return (pa["edge"]["W"].T,
                pa["edge"]["b"] + pa["key"]["b"] + pa["query"]["b"])

    e = None
    for li, (pa, pb) in enumerate(zip(atom_layers, params["bond_layers"])):
        wkqvs = jnp.concatenate(
            [pa["key"]["W"], pa["query"]["W"], pa["value"]["W"],
             pa["skip"]["W"]], axis=0).T
        bkqvs = jnp.concatenate(
            [jnp.zeros((_D,), jnp.float32), jnp.zeros((_D,), jnp.float32),
             pa["value"]["b"], pa["skip"]["b"]])
        kskip, qv = _atom_proj(x, wkqvs, bkqvs, br=2000)
        kk, skip_o = kskip[:, :_D], kskip[:, _D:]
        if e is None:
            wet, ebias = ew_bias(pa)
            e = _linear(edge_attr, wet, ebias, br=4000)
        msg = _sc_atom_msg(kk, qv, e, dst, src)
        agg = _sc_scatter_add(msg, dst, zrows)
        x = _atom_post(x, skip_o, agg, pa["bn_gamma"], pa["bn_beta"])

        w12 = jnp.concatenate([pb["v1"]["W"], pb["v2"]["W"]], axis=0).T
        a12 = _linear(x, w12, None, br=2000)
        a1, a2 = a12[:, :_D], a12[:, _D:]
        b0 = pb["v0"]["b"] + pb["v1"]["b"] + pb["v2"]["b"]
        a1g, a2g = _sc_gather((a1, a2), (src, dst), (0, 1))
        s, stats = _bond_sum_stats(edge_attr, pb["v0"]["W"].T, b0, a1g, a2g)
        if li + 1 < n_layers:
            wet, ebias = ew_bias(atom_layers[li + 1])
            edge_attr, e = _bond_post(edge_attr, s, stats, pb["bn_gamma"],
                                      pb["bn_beta"], wet, ebias)
        else:
            edge_attr = _bond_post(edge_attr, s, stats, pb["bn_gamma"],
                                   pb["bn_beta"])

    boa = _atom_head(x, params["atom_mlp"]["l1"]["W"].T,
                     params["atom_mlp"]["l1"]["b"],
                     params["atom_mlp"]["l2"]["W"].T,
                     params["atom_mlp"]["l2"]["b"])
    bonds = _bonds_head(edge_attr, params["bond_mlp"]["l1"]["W"].T,
                        params["bond_mlp"]["l1"]["b"],
                        params["bond_mlp"]["l2"]["W"].T,
                        params["bond_mlp"]["l2"]["b"])
    return boa.reshape(-1, 8, 100), bonds


# speedup vs baseline: 4.0144x; 1.0383x over previous
import functools

import jax
import jax.numpy as jnp
from jax import lax
from jax.experimental import pallas as pl
from jax.experimental.pallas import tpu as pltpu
from jax.experimental.pallas import tpu_sc as plsc

_N = 10000
_E = 160000
_D = 128
_NC = 2
_NS = 16
_NW = _NC * _NS
_RPW = _E // _NW
_CH = 128
_NFULL = _RPW // _CH
_TAIL = _RPW - _NFULL * _CH
_NPAD = 10240
_RPT = _NPAD // _NS


def _sc_mesh():
    return plsc.VectorSubcoreMesh(core_axis_name="c", subcore_axis_name="s",
                                  num_cores=_NC, num_subcores=_NS)


_NBUF = 4
_SNBUF = 3
_SCH = 96
_SCHUNKS = ([(c * _SCH, _SCH) for c in range(_RPW // _SCH)]
            + [((_RPW // _SCH) * _SCH, _RPW - (_RPW // _SCH) * _SCH)])
_CHUNKS = [(c * _CH, _CH) for c in range(_NFULL)] + [(_NFULL * _CH, _TAIL)]


def _sc_gather(tables, idxs, idx_of):
    n_t = len(tables)
    n_i = len(idxs)
    width = tables[0].shape[1]
    dt = tables[0].dtype
    scratch = (
        [pltpu.VMEM((_RPW,), jnp.int32) for _ in range(n_i)]
        + [pltpu.VMEM((_CH, width), dt) for _ in range(_NBUF)]
        + [pltpu.SemaphoreType.DMA, pltpu.SemaphoreType.DMA]
    )
    out_type = tuple(
        jax.ShapeDtypeStruct((_E, width), dt) for _ in range(n_t))

    def body(*refs):
        tab = refs[:n_t]
        idx = refs[n_t:n_t + n_i]
        outs = refs[n_t + n_i:n_t + n_i + n_t]
        s = n_t + n_i + n_t
        ib = refs[s:s + n_i]; s += n_i
        rb = refs[s:s + _NBUF]; s += _NBUF
        sem_g, sem_w = refs[s], refs[s + 1]
        wid = lax.axis_index("s") * _NC + lax.axis_index("c")
        base = wid * _RPW
        for j in range(n_i):
            pltpu.sync_copy(idx[j].at[pl.ds(base, _RPW)], ib[j])

        jobs = [(t, off, sz) for t in range(n_t) for (off, sz) in _CHUNKS]
        pend_w = [None] * _NBUF
        prev = None

        def buf(b, sz):
            return rb[b] if sz == _CH else rb[b].at[pl.ds(0, sz)]

        def flush_prev():
            pb, pt, poff, psz, pg = prev
            pg.wait()
            pend_w[pb] = pltpu.async_copy(
                buf(pb, psz), outs[pt].at[pl.ds(base + poff, psz)], sem_w)

        for c, (t, off, sz) in enumerate(jobs):
            b = c % _NBUF
            if pend_w[b] is not None:
                pend_w[b].wait()
                pend_w[b] = None
            g = pltpu.async_copy(
                tab[t].at[ib[idx_of[t]].at[pl.ds(off, sz)]], buf(b, sz), sem_g)
            if prev is not None:
                flush_prev()
            prev = (b, t, off, sz, g)
        flush_prev()
        for d in pend_w:
            if d is not None:
                d.wait()

    f = pl.kernel(body, out_type=out_type, mesh=_sc_mesh(),
                  scratch_types=scratch)
    return f(*tables, *idxs)


def _sc_atom_msg(kk, qv, e, dst, src):
    scratch = (
        [pltpu.VMEM((_RPW,), jnp.int32) for _ in range(2)]
        + [pltpu.VMEM((_CH, _D), jnp.float32) for _ in range(2)]
        + [pltpu.VMEM((_CH, _D), jnp.int32) for _ in range(2)]
        + [pltpu.VMEM((_CH, _D), jnp.float32) for _ in range(2)]
        + [pltpu.SemaphoreType.DMA for _ in range(4)]
    )

    def body(kk_h, qv_h, e_h, dst_h, src_h, out_h, *rest):
        ibd, ibs = rest[0], rest[1]
        bk = rest[2:4]
        bqv = rest[4:6]
        be = rest[6:8]
        sem_g = rest[8:10]
        sem_w = rest[10:12]
        wid = lax.axis_index("s") * _NC + lax.axis_index("c")
        base = wid * _RPW
        pltpu.sync_copy(dst_h.at[pl.ds(base, _RPW)], ibd)
        pltpu.sync_copy(src_h.at[pl.ds(base, _RPW)], ibs)

        def sl(ref, sz):
            return ref if sz == _CH else ref.at[pl.ds(0, sz)]

        pend_w = [None, None]
        prev = None

        def compute_and_flush():
            b, off, sz, descs = prev
            for d_ in descs:
                d_.wait()

            def row(r, carry):
                for j in range(_D // 16):
                    s_ = pl.ds(j * 16, 16)
                    x_ = bqv[b][r, s_]
                    qf = ((x_ << 16) >> 16).astype(jnp.float32) * (1.0 / 256.0)
                    vf = (x_ >> 16).astype(jnp.float32) * (1.0 / 256.0)
                    xv = bk[b][r, s_] + qf + be[b][r, s_]
                    be[b][r, s_] = vf / (1.0 + jnp.exp(-xv))
                return carry

            lax.fori_loop(0, sz, row, 0)
            pend_w[b] = pltpu.async_copy(
                sl(be[b], sz), out_h.at[pl.ds(base + off, sz)], sem_w[b])

        for c, (off, sz) in enumerate(_CHUNKS):
            b = c % 2
            if pend_w[b] is not None:
                pend_w[b].wait()
                pend_w[b] = None
            ds_ = ibd.at[pl.ds(off, sz)]
            ss_ = ibs.at[pl.ds(off, sz)]
            descs = [
                pltpu.async_copy(kk_h.at[ds_], sl(bk[b], sz), sem_g[b]),
                pltpu.async_copy(qv_h.at[ss_], sl(bqv[b], sz), sem_g[b]),
                pltpu.async_copy(e_h.at[pl.ds(base + off, sz)],
                                 sl(be[b], sz), sem_g[b]),
            ]
            if prev is not None:
                compute_and_flush()
            prev = (b, off, sz, descs)
        compute_and_flush()
        for d_ in pend_w:
            if d_ is not None:
                d_.wait()

    f = pl.kernel(body, out_type=jax.ShapeDtypeStruct((_E, _D), jnp.float32),
                  mesh=_sc_mesh(), scratch_types=scratch)
    return f(kk, qv, e, dst, src)


def _sc_scatter_add(msg, dst, zrows):
    scratch = (
        [pltpu.VMEM((_SCH,), jnp.int32) for _ in range(_SNBUF)]
        + [pltpu.VMEM((_SCH, _D), jnp.float32) for _ in range(_SNBUF)]
        + [pltpu.VMEM((_RPW - (_RPW // _SCH) * _SCH,), jnp.int32),
           pltpu.VMEM((_RPW - (_RPW // _SCH) * _SCH, _D), jnp.float32),
           pltpu.VMEM_SHARED((_NPAD, _D), jnp.float32),
           pltpu.SemaphoreType.DMA, pltpu.SemaphoreType.DMA]
    )

    def body(msg_hbm, dst_hbm, z_hbm, out_hbm, *rest):
        ib = rest[:_SNBUF]
        rb = rest[_SNBUF:2 * _SNBUF]
        ib_t, rb_t, acc, sem_l, sem_s = rest[2 * _SNBUF:]
        cid = lax.axis_index("c")
        sid = lax.axis_index("s")
        wid = sid * _NC + cid
        base = wid * _RPW
        pltpu.sync_copy(z_hbm, acc.at[pl.ds(sid * _RPT, _RPT)])
        plsc.subcore_barrier()

        pend_s = [None] * (_SNBUF + 1)
        prev = None

        def flush_prev():
            slot, pib, prb, li, lm = prev
            li.wait()
            lm.wait()
            pend_s[slot] = pltpu.async_copy(prb, acc.at[pib], sem_s, add=True)

        for c, (off, sz) in enumerate(_SCHUNKS):
            if sz == _SCH:
                slot = c % _SNBUF
                cib, crb = ib[slot], rb[slot]
            else:
                slot = _SNBUF
                cib, crb = ib_t, rb_t
            if pend_s[slot] is not None:
                pend_s[slot].wait()
                pend_s[slot] = None
            li = pltpu.async_copy(dst_hbm.at[pl.ds(base + off, sz)], cib, sem_l)
            lm = pltpu.async_copy(msg_hbm.at[pl.ds(base + off, sz)], crb, sem_l)
            if prev is not None:
                flush_prev()
            prev = (slot, cib, crb, li, lm)
        flush_prev()
        for d in pend_s:
            if d is not None:
                d.wait()
        plsc.subcore_barrier()
        pltpu.sync_copy(acc.at[pl.ds(sid * _RPT, _RPT)],
                        out_hbm.at[cid, pl.ds(sid * _RPT, _RPT)])

    f = pl.kernel(body,
                  out_type=jax.ShapeDtypeStruct((_NC, _NPAD, _D), jnp.float32),
                  mesh=_sc_mesh(), scratch_types=scratch)
    return f(msg, dst, zrows)


def _linear(x, wt, b=None, act=None, br=4000, out_dtype=jnp.float32):
    r, d = x.shape
    f_out = wt.shape[1]
    assert r % br == 0
    grid = (r // br,)
    in_specs = [pl.BlockSpec((br, d), lambda i: (i, 0)),
                pl.BlockSpec((d, f_out), lambda i: (0, 0))]
    args = [x, wt]
    if b is not None:
        in_specs.append(pl.BlockSpec((1, f_out), lambda i: (0, 0)))
        args.append(b.reshape(1, f_out))

    def body(x_ref, w_ref, *rest):
        if b is not None:
            b_ref, o_ref = rest
        else:
            (o_ref,) = rest
        y = jnp.dot(x_ref[...], w_ref[...], preferred_element_type=jnp.float32)
        if b is not None:
            y = y + b_ref[...]
        if act == "relu":
            y = jnp.maximum(y, 0.0)
        o_ref[...] = y.astype(out_dtype)

    return pl.pallas_call(
        body, grid=grid, in_specs=in_specs,
        out_specs=pl.BlockSpec((br, f_out), lambda i: (i, 0)),
        out_shape=jax.ShapeDtypeStruct((r, f_out), out_dtype))(*args)


def _atom_post(x, skip_o, agg, gamma, beta):
    def body(x_r, s_r, a_r, g_r, b_r, o_r):
        h = s_r[...] + a_r[0, :_N] + a_r[1, :_N]
        mu = jnp.mean(h, axis=0, keepdims=True)
        var = jnp.mean((h - mu) ** 2, axis=0, keepdims=True)
        hn = g_r[...] * (h - mu) * lax.rsqrt(var + 1e-5) + b_r[...]
        o_r[...] = x_r[...] + jnp.maximum(hn, 0.0)

    return pl.pallas_call(
        body, out_shape=jax.ShapeDtypeStruct((_N, _D), jnp.float32))(
            x, skip_o, agg, gamma.reshape(1, _D), beta.reshape(1, _D))


def _bond_sum_stats(ea, wt, b, a1g, a2g, br=4000):
    grid = (_E // br,)
    spec = pl.BlockSpec((br, _D), lambda i: (i, 0))
    wspec = pl.BlockSpec((_D, _D), lambda i: (0, 0))
    bspec = pl.BlockSpec((1, _D), lambda i: (0, 0))

    def body(ea_r, w_r, b_r, a1_r, a2_r, s_r, st_r, sacc, qacc):
        i = pl.program_id(0)
        v = (jnp.dot(ea_r[...], w_r[...], preferred_element_type=jnp.float32)
             + b_r[...] + a1_r[...] + a2_r[...])
        s_r[...] = v.astype(jnp.bfloat16)

        @pl.when(i == 0)
        def _():
            sacc[...] = jnp.zeros_like(sacc)
            qacc[...] = jnp.zeros_like(qacc)

        sacc[...] += jnp.sum(v, axis=0).reshape(1, _D)
        qacc[...] += jnp.sum(v * v, axis=0).reshape(1, _D)

        @pl.when(i == grid[0] - 1)
        def _():
            st_r[0:1, :] = sacc[...]
            st_r[1:2, :] = qacc[...]

    return pl.pallas_call(
        body, grid=grid, in_specs=[spec, wspec, bspec, spec, spec],
        out_specs=(spec, pl.BlockSpec((2, _D), lambda i: (0, 0))),
        out_shape=(jax.ShapeDtypeStruct((_E, _D), jnp.bfloat16),
                   jax.ShapeDtypeStruct((2, _D), jnp.float32)),
        scratch_shapes=[pltpu.VMEM((1, _D), jnp.float32),
                        pltpu.VMEM((1, _D), jnp.float32)])(
            ea, wt, b.reshape(1, _D), a1g, a2g)


def _bond_post(ea, s, stats, gamma, beta, we_next=None, be_next=None,
               br=4000):
    grid = (_E // br,)
    spec = pl.BlockSpec((br, _D), lambda i: (i, 0))
    one = pl.BlockSpec((1, _D), lambda i: (0, 0))
    fused = we_next is not None

    def body(ea_r, s_r, st_r, g_r, b_r, *rest):
        mu = st_r[0:1, :] * (1.0 / _E)
        var = st_r[1:2, :] * (1.0 / _E) - mu * mu
        hn = (g_r[...] * (s_r[...].astype(jnp.float32) - mu)
              * lax.rsqrt(var + 1e-5) + b_r[...])
        ea_new = ea_r[...] + jnp.maximum(hn, 0.0)
        if fused:
            w_r, eb_r, o_r, e_r = rest
            e_r[...] = jnp.dot(ea_new, w_r[...],
                               preferred_element_type=jnp.float32) + eb_r[...]
        else:
            (o_r,) = rest
        o_r[...] = ea_new

    in_specs = [spec, spec, pl.BlockSpec((2, _D), lambda i: (0, 0)), one, one]
    args = [ea, s, stats, gamma.reshape(1, _D), beta.reshape(1, _D)]
    if fused:
        in_specs += [pl.BlockSpec((_D, _D), lambda i: (0, 0)), one]
        args += [we_next, be_next.reshape(1, _D)]
        return pl.pallas_call(
            body, grid=grid, in_specs=in_specs, out_specs=(spec, spec),
            out_shape=(jax.ShapeDtypeStruct((_E, _D), jnp.float32),
                       jax.ShapeDtypeStruct((_E, _D), jnp.float32)))(*args)
    return pl.pallas_call(
        body, grid=grid, in_specs=in_specs, out_specs=spec,
        out_shape=jax.ShapeDtypeStruct((_E, _D), jnp.float32))(*args)


def _atom_proj(x, wt, b, br=4000):
    grid = (_N // br,)

    def body(x_ref, w_ref, b_ref, ks_ref, qv_ref):
        y = jnp.dot(x_ref[...], w_ref[...],
                    preferred_element_type=jnp.float32) + b_ref[...]
        ks_ref[:, :_D] = y[:, :_D]
        ks_ref[:, _D:] = y[:, 3 * _D:]
        qi = jnp.clip(jnp.round(y[:, _D:2 * _D] * 256.0), -32768.0,
                      32767.0).astype(jnp.int32)
        vi = jnp.clip(jnp.round(y[:, 2 * _D:3 * _D] * 256.0), -32768.0,
                      32767.0).astype(jnp.int32)
        qv_ref[...] = (vi << 16) | (qi & 0xFFFF)

    return pl.pallas_call(
        body, grid=grid,
        in_specs=[pl.BlockSpec((br, _D), lambda i: (i, 0)),
                  pl.BlockSpec((_D, 4 * _D), lambda i: (0, 0)),
                  pl.BlockSpec((1, 4 * _D), lambda i: (0, 0))],
        out_specs=(pl.BlockSpec((br, 2 * _D), lambda i: (i, 0)),
                   pl.BlockSpec((br, _D), lambda i: (i, 0))),
        out_shape=(jax.ShapeDtypeStruct((_N, 2 * _D), jnp.float32),
                   jax.ShapeDtypeStruct((_N, _D), jnp.int32)))(
            x, wt, b.reshape(1, 4 * _D))


def _bonds_head(ea, w1t, b1, w2t, b2, br=4000):
    grid = (_E // br,)
    f_out = w2t.shape[1]
    spec = pl.BlockSpec((br, _D), lambda i: (i, 0))

    def body(ea_r, w1_r, b1_r, w2_r, b2_r, o_r):
        h = jnp.maximum(
            jnp.dot(ea_r[...], w1_r[...], preferred_element_type=jnp.float32)
            + b1_r[...], 0.0)
        o_r[...] = jnp.dot(h, w2_r[...],
                           preferred_element_type=jnp.float32) + b2_r[...]

    return pl.pallas_call(
        body, grid=grid,
        in_specs=[spec, pl.BlockSpec((_D, _D), lambda i: (0, 0)),
                  pl.BlockSpec((1, _D), lambda i: (0, 0)),
                  pl.BlockSpec((_D, f_out), lambda i: (0, 0)),
                  pl.BlockSpec((1, f_out), lambda i: (0, 0))],
        out_specs=pl.BlockSpec((br, f_out), lambda i: (i, 0)),
        out_shape=jax.ShapeDtypeStruct((_E, f_out), jnp.float32))(
            ea, w1t, b1.reshape(1, _D), w2t, b2.reshape(1, f_out))


def _atom_head(x, w1t, b1, w2t, b2):
    f_out = w2t.shape[1]

    def body(x_r, w1_r, b1_r, w2_r, b2_r, o_r):
        z = jnp.mean(x_r[...], axis=0, keepdims=True)
        h = jnp.maximum(
            jnp.dot(z, w1_r[...], preferred_element_type=jnp.float32)
            + b1_r[...], 0.0)
        o_r[...] = jnp.dot(h, w2_r[...],
                           preferred_element_type=jnp.float32) + b2_r[...]

    return pl.pallas_call(
        body, out_shape=jax.ShapeDtypeStruct((1, f_out), jnp.float32))(
            x, w1t, b1.reshape(1, -1), w2t, b2.reshape(1, -1))


def kernel(x, edge_index, edge_attr, params):
    src = edge_index[0]
    dst = edge_index[1]
    zrows = jnp.zeros((_RPT, _D), jnp.float32)

    atom_layers = params["atom_layers"]
    n_layers = len(atom_layers)

    def ew_bias(pa):
        return (pa["edge"]["W"].T,
                pa["edge"]["b"] + pa["key"]["b"] + pa["query"]["b"])

    e = None
    for li, (pa, pb) in enumerate(zip(atom_layers, params["bond_layers"])):
        wkqvs = jnp.concatenate(
            [pa["key"]["W"], pa["query"]["W"], pa["value"]["W"],
             pa["skip"]["W"]], axis=0).T
        bkqvs = jnp.concatenate(
            [jnp.zeros((_D,), jnp.float32), jnp.zeros((_D,), jnp.float32),
             pa["value"]["b"], pa["skip"]["b"]])
        kskip, qv = _atom_proj(x, wkqvs, bkqvs, br=2000)
        kk, skip_o = kskip[:, :_D], kskip[:, _D:]
        if e is None:
            wet, ebias = ew_bias(pa)
            e = _linear(edge_attr, wet, ebias, br=4000)
        msg = _sc_atom_msg(kk, qv, e, dst, src)
        agg = _sc_scatter_add(msg, dst, zrows)
        x = _atom_post(x, skip_o, agg, pa["bn_gamma"], pa["bn_beta"])

        w12 = jnp.concatenate([pb["v1"]["W"], pb["v2"]["W"]], axis=0).T
        a12 = _linear(x, w12, None, br=2000)
        a1, a2 = a12[:, :_D], a12[:, _D:]
        b0 = pb["v0"]["b"] + pb["v1"]["b"] + pb["v2"]["b"]
        a1g, a2g = _sc_gather((a1, a2), (src, dst), (0, 1))
        s, stats = _bond_sum_stats(edge_attr, pb["v0"]["W"].T, b0, a1g, a2g)
        if li + 1 < n_layers:
            wet, ebias = ew_bias(atom_layers[li + 1])
            edge_attr, e = _bond_post(edge_attr, s, stats, pb["bn_gamma"],
                                      pb["bn_beta"], wet, ebias)
        else:
            edge_attr = _bond_post(edge_attr, s, stats, pb["bn_gamma"],
                                   pb["bn_beta"])

    boa = _atom_head(x, params["atom_mlp"]["l1"]["W"].T,
                     params["atom_mlp"]["l1"]["b"],
                     params["atom_mlp"]["l2"]["W"].T,
                     params["atom_mlp"]["l2"]["b"])
    bonds = _bonds_head(edge_attr, params["bond_mlp"]["l1"]["W"].T,
                        params["bond_mlp"]["l1"]["b"],
                        params["bond_mlp"]["l2"]["W"].T,
                        params["bond_mlp"]["l2"]["b"])
    return boa.reshape(-1, 8, 100), bonds
